# Initial kernel scaffold; baseline (speedup 1.0000x reference)
import jax
import jax.numpy as jnp
from jax.experimental import pallas as pl


def kernel(node, edge_attr, pdd, edge_index, batch, Wa1, ba1, Wa2, ba2, Wemb, bemb, Wp, bp, Wq, bq, Wk, bk, Wv, bv, Wed, bed, Ws, bs, W1p, b1p, W2p, b2p, W3p, b3p, gbn, bbn, Wf, bf, Wo, bo):
    raise NotImplementedError("write your pallas kernel here")


# speedup vs baseline: 1.9142x; 1.9142x over previous
import functools

import jax
import jax.numpy as jnp
from jax import lax
from jax.experimental import pallas as pl
from jax.experimental.pallas import tpu as pltpu
from jax.experimental.pallas import tpu_sc as plsc

N = 10000
E = 160000
G = 64
D = 256

BE = 640
NEB = E // BE
BN = 400
NNB = N // BN

NC = 2
NS = 16
NW = NC * NS
CE = E // NW
GB = 200
NGC = CE // GB
SC_C = 2000
SC_NCH = E // SC_C
NR = 320
DUMP = NR
ACCR = NR + 8
_f32 = jnp.float32


def _silu(x):
    return x * jax.nn.sigmoid(x)


def _embed_body(ef_ref, wemb_ref, bemb_ref, wed_ref, bed_ref, e0_ref, e1_ref, e2_ref):
    ef = ef_ref[...]
    centers = -6.0 + (6.0 / 255.0) * lax.broadcasted_iota(jnp.int32, (1, 256), 1).astype(_f32)
    gamma = 1.0 / (6.0 / 255.0)
    diff = ef - centers
    rbf = jnp.exp(-gamma * diff * diff)
    z = jnp.dot(rbf, wemb_ref[...], preferred_element_type=_f32) + bemb_ref[...]
    e = _silu(z)
    ee = jnp.dot(e, wed_ref[...], preferred_element_type=_f32) + bed_ref[...]
    e0_ref[...] = ee[:, :256]
    e1_ref[...] = ee[:, 256:512]
    e2_ref[...] = ee[:, 512:]


def _embed(ef3d, wemb, bemb, wedcat, bedcat):
    return pl.pallas_call(
        _embed_body,
        grid=(NEB,),
        in_specs=[
            pl.BlockSpec((BE, 1), lambda i: (i, 0)),
            pl.BlockSpec((256, 256), lambda i: (0, 0)),
            pl.BlockSpec((1, 256), lambda i: (0, 0)),
            pl.BlockSpec((256, 768), lambda i: (0, 0)),
            pl.BlockSpec((1, 768), lambda i: (0, 0)),
        ],
        out_specs=[pl.BlockSpec((BE, 256), lambda i: (i, 0))] * 3,
        out_shape=[jax.ShapeDtypeStruct((E, 256), _f32)] * 3,
        compiler_params=pltpu.CompilerParams(dimension_semantics=("parallel",)),
    )(ef3d, wemb, bemb, wedcat, bedcat)


def _x0p0_body(node_ref, pdd_ref, wa1_ref, ba1_ref, wa2_ref, ba2_ref,
               wp_ref, bp_ref, x0_ref, p0_ref):
    h = jnp.dot(node_ref[...], wa1_ref[...], preferred_element_type=_f32) + ba1_ref[...]
    h = _silu(h)
    x0_ref[...] = jnp.dot(h, wa2_ref[...], preferred_element_type=_f32) + ba2_ref[...]
    p0_ref[...] = jnp.dot(pdd_ref[...], wp_ref[...], preferred_element_type=_f32) + bp_ref[...]


def _x0p0(nodep, pddp, wa1p, ba1, wa2, ba2, wpp, bp):
    return pl.pallas_call(
        _x0p0_body,
        grid=(NNB,),
        in_specs=[
            pl.BlockSpec((BN, 128), lambda i: (i, 0)),
            pl.BlockSpec((BN, 128), lambda i: (i, 0)),
            pl.BlockSpec((128, 256), lambda i: (0, 0)),
            pl.BlockSpec((1, 256), lambda i: (0, 0)),
            pl.BlockSpec((256, 256), lambda i: (0, 0)),
            pl.BlockSpec((1, 256), lambda i: (0, 0)),
            pl.BlockSpec((128, 256), lambda i: (0, 0)),
            pl.BlockSpec((1, 256), lambda i: (0, 0)),
        ],
        out_specs=[pl.BlockSpec((BN, 256), lambda i: (i, 0))] * 2,
        out_shape=[jax.ShapeDtypeStruct((N, 256), _f32)] * 2,
        compiler_params=pltpu.CompilerParams(dimension_semantics=("parallel",)),
    )(nodep, pddp, wa1p, ba1, wa2, ba2, wpp, bp)


def _qkvs_body(x_ref, w_ref, b_ref, q_ref, k_ref, v_ref, s_ref):
    o = jnp.dot(x_ref[...], w_ref[...], preferred_element_type=_f32) + b_ref[...]
    q_ref[...] = o[:, :256]
    k_ref[...] = o[:, 256:512]
    v_ref[...] = o[:, 512:768]
    s_ref[...] = o[:, 768:]


def _qkvs(x, wcat, bcat):
    return pl.pallas_call(
        _qkvs_body,
        grid=(NNB,),
        in_specs=[
            pl.BlockSpec((BN, 256), lambda i: (i, 0)),
            pl.BlockSpec((256, 1024), lambda i: (0, 0)),
            pl.BlockSpec((1, 1024), lambda i: (0, 0)),
        ],
        out_specs=[pl.BlockSpec((BN, 256), lambda i: (i, 0))] * 4,
        out_shape=[jax.ShapeDtypeStruct((N, 256), _f32)] * 4,
        compiler_params=pltpu.CompilerParams(dimension_semantics=("parallel",)),
    )(x, wcat, bcat)


@functools.cache
def _sc_gather_fn():
    mesh = plsc.VectorSubcoreMesh(core_axis_name="c", subcore_axis_name="s")
    return functools.partial(
        pl.kernel,
        mesh=mesh,
        out_type=[jax.ShapeDtypeStruct((E, 256), _f32)] * 3,
        scratch_types=[
            pltpu.VMEM((GB,), jnp.int32),
            pltpu.VMEM((GB,), jnp.int32),
            pltpu.VMEM((GB, 256), _f32),
            pltpu.SemaphoreType.DMA,
        ],
    )(_sc_gather_body)


def _sc_gather_body(src_hbm, dst_hbm, q_hbm, k_hbm, v_hbm,
                    qd_hbm, ks_hbm, vs_hbm, srci, dsti, rows, sem):
    c = lax.axis_index("c")
    s = lax.axis_index("s")
    wid = s * NC + c
    base0 = wid * CE

    def body(j, carry):
        base = base0 + j * GB
        pltpu.sync_copy(src_hbm.at[pl.ds(base, GB)], srci)
        pltpu.sync_copy(dst_hbm.at[pl.ds(base, GB)], dsti)
        pltpu.async_copy(k_hbm.at[srci], rows, sem).wait()
        pltpu.sync_copy(rows, ks_hbm.at[pl.ds(base, GB)])
        pltpu.async_copy(v_hbm.at[srci], rows, sem).wait()
        pltpu.sync_copy(rows, vs_hbm.at[pl.ds(base, GB)])
        pltpu.async_copy(q_hbm.at[dsti], rows, sem).wait()
        pltpu.sync_copy(rows, qd_hbm.at[pl.ds(base, GB)])
        return carry

    lax.fori_loop(0, NGC, body, 0)


def _alpha_body(qd_ref, ks_ref, ee_ref, a_ref, m_ref):
    a = jnp.sum(qd_ref[...] * (ks_ref[...] + ee_ref[...]), axis=1, keepdims=True) / 16.0
    a_ref[...] = a
    m_ref[...] = jnp.full((1, 1, 128), jnp.max(a), _f32)


def _alpha(qd, ks, ee):
    return pl.pallas_call(
        _alpha_body,
        grid=(NEB,),
        in_specs=[pl.BlockSpec((BE, 256), lambda i: (i, 0))] * 3,
        out_specs=[
            pl.BlockSpec((BE, 1), lambda i: (i, 0)),
            pl.BlockSpec((1, 1, 128), lambda i: (i, 0, 0)),
        ],
        out_shape=[
            jax.ShapeDtypeStruct((E, 1), _f32),
            jax.ShapeDtypeStruct((NEB, 1, 128), _f32),
        ],
        compiler_params=pltpu.CompilerParams(dimension_semantics=("parallel",)),
    )(qd, ks, ee)


def _y_body(c_ref, a_ref, vs_ref, ee_ref, yn_ref, ye_ref):
    ex = jnp.exp(a_ref[...] - c_ref[0, 0])
    ye_ref[...] = ex
    yn_ref[...] = (vs_ref[...] + ee_ref[...]) * ex


def _y(cmax, alpha, vs, ee):
    return pl.pallas_call(
        _y_body,
        grid=(NEB,),
        in_specs=[
            pl.BlockSpec(memory_space=pltpu.SMEM),
            pl.BlockSpec((BE, 1), lambda i: (i, 0)),
            pl.BlockSpec((BE, 256), lambda i: (i, 0)),
            pl.BlockSpec((BE, 256), lambda i: (i, 0)),
        ],
        out_specs=[
            pl.BlockSpec((BE, 256), lambda i: (i, 0)),
            pl.BlockSpec((BE, 1), lambda i: (i, 0)),
        ],
        out_shape=[
            jax.ShapeDtypeStruct((E, 256), _f32),
            jax.ShapeDtypeStruct((E, 1), _f32),
        ],
        compiler_params=pltpu.CompilerParams(dimension_semantics=("parallel",)),
    )(cmax, alpha, vs, ee)


@functools.cache
def _sc_scatter_fn():
    mesh = plsc.VectorSubcoreMesh(core_axis_name="c", subcore_axis_name="s")
    return functools.partial(
        pl.kernel,
        mesh=mesh,
        compiler_params=pltpu.CompilerParams(needs_layout_passes=False),
        out_type=[
            jax.ShapeDtypeStruct((NW, NR * 256), _f32),
            jax.ShapeDtypeStruct((NW, NR * 16), _f32),
        ],
        scratch_types=[
            pltpu.VMEM((SC_C,), jnp.int32),
            pltpu.VMEM((SC_C,), _f32),
            pltpu.VMEM((SC_C + 16,), jnp.int32),
            pltpu.VMEM((SC_C + 16,), jnp.int32),
            pltpu.VMEM((SC_C + 16,), _f32),
            pltpu.VMEM((16, 256), _f32),
            pltpu.VMEM((ACCR * 256,), _f32),
            pltpu.VMEM((ACCR * 16,), _f32),
            pltpu.SemaphoreType.DMA,
        ],
    )(_sc_scatter_body)


def _sc_scatter_body(yn_hbm, ye_hbm, dst_hbm, zn_hbm, ze_hbm, on_hbm, od_hbm,
                     dbuf, xbuf, ebuf, lbuf, cxbuf, yrow, accn, acce, sem):
    c = lax.axis_index("c")
    s = lax.axis_index("s")
    w = s * NC + c
    lo = w * NR
    lane = lax.iota(jnp.int32, 16)

    pltpu.sync_copy(zn_hbm, accn)
    pltpu.sync_copy(ze_hbm, acce)

    def chunk_body(j, carry):
        base = j * SC_C
        pltpu.sync_copy(dst_hbm.at[pl.ds(base, SC_C)], dbuf)
        pltpu.sync_copy(ye_hbm.at[pl.ds(base, SC_C)], xbuf)

        def scan_body(g, f):
            d16 = dbuf[pl.ds(g * 16, 16)]
            loc = d16 - lo
            ok = (loc >= 0) & (loc < NR)
            eid = base + g * 16 + lane
            plsc.store_compressed(ebuf.at[pl.ds(f, 16)], eid, mask=ok)
            plsc.store_compressed(lbuf.at[pl.ds(f, 16)], loc, mask=ok)
            plsc.store_compressed(cxbuf.at[pl.ds(f, 16)], xbuf[pl.ds(g * 16, 16)], mask=ok)
            cnt = plsc.all_reduce_population_count(ok)
            return f + cnt[0]

        f = lax.fori_loop(0, SC_C // 16, scan_body, 0)
        ebuf[pl.ds(f, 16)] = jnp.zeros((16,), jnp.int32)
        lbuf[pl.ds(f, 16)] = jnp.full((16,), DUMP, jnp.int32)
        cxbuf[pl.ds(f, 16)] = jnp.zeros((16,), _f32)
        n_units = (f + 15) // 16

        def unit_body(u, carry2):
            pltpu.async_copy(yn_hbm.at[ebuf.at[pl.ds(u * 16, 16)]], yrow, sem).wait()

            def row_body(r, carry3):
                pos = jnp.full((16,), u * 16 + r, jnp.int32)
                ii = plsc.load_gather(lbuf, [pos])
                exv = plsc.load_gather(cxbuf, [pos])
                basea = ii * 256
                for l in range(16):
                    vals = yrow[r, pl.ds(l * 16, 16)]
                    plsc.addupdate_scatter(accn, [basea + (l * 16) + lane], vals)
                plsc.addupdate_scatter(acce, [ii * 16 + lane], exv)
                return carry3

            lax.fori_loop(0, 16, row_body, 0)
            return carry2

        lax.fori_loop(0, n_units, unit_body, 0)
        return carry

    lax.fori_loop(0, SC_NCH, chunk_body, 0)

    pltpu.sync_copy(accn.at[pl.ds(0, NR * 256)], on_hbm.at[w])
    pltpu.sync_copy(acce.at[pl.ds(0, NR * 16)], od_hbm.at[w])


def _pdd1_body(num_ref, den_ref, xs_ref, p_ref, x_ref, adj_ref, s1_ref, s2_ref):
    den = jnp.max(den_ref[...], axis=1, keepdims=True)
    x = num_ref[...] * (1.0 / (den + 1e-16)) + xs_ref[...]
    adj = p_ref[...] + x
    x_ref[...] = x
    adj_ref[...] = adj
    s1_ref[...] = jnp.sum(adj, axis=0, keepdims=True).reshape(1, 1, 256)
    s2_ref[...] = jnp.sum(adj * adj, axis=0, keepdims=True).reshape(1, 1, 256)


def _pdd1(num, den, xs, p):
    return pl.pallas_call(
        _pdd1_body,
        grid=(NNB,),
        in_specs=[
            pl.BlockSpec((BN, 256), lambda i: (i, 0)),
            pl.BlockSpec((BN, 16), lambda i: (i, 0)),
            pl.BlockSpec((BN, 256), lambda i: (i, 0)),
            pl.BlockSpec((BN, 256), lambda i: (i, 0)),
        ],
        out_specs=[
            pl.BlockSpec((BN, 256), lambda i: (i, 0)),
            pl.BlockSpec((BN, 256), lambda i: (i, 0)),
            pl.BlockSpec((1, 1, 256), lambda i: (i, 0, 0)),
            pl.BlockSpec((1, 1, 256), lambda i: (i, 0, 0)),
        ],
        out_shape=[
            jax.ShapeDtypeStruct((N, 256), _f32),
            jax.ShapeDtypeStruct((N, 256), _f32),
            jax.ShapeDtypeStruct((NNB, 1, 256), _f32),
            jax.ShapeDtypeStruct((NNB, 1, 256), _f32),
        ],
        compiler_params=pltpu.CompilerParams(dimension_semantics=("parallel",)),
    )(num, den, xs, p)


def _pdd2_body(adj_ref, x_ref, sc_ref, sh_ref, w1_ref, b1_ref, w2_ref, b2_ref,
               w3_ref, b3_ref, o_ref):
    h = adj_ref[...] * sc_ref[...] + sh_ref[...]
    h2 = jnp.dot(h, w1_ref[...], preferred_element_type=_f32) + b1_ref[...]
    x1 = h2[:, :256]
    x2 = h2[:, 256:]
    x1 = jnp.dot(x1, w2_ref[...], preferred_element_type=_f32) + b2_ref[...]
    x2 = 0.5 * x2 * (1.0 + lax.erf(x2 * 0.7071067811865476))
    o_ref[...] = (jnp.dot(x1 * x2, w3_ref[...], preferred_element_type=_f32)
                  + b3_ref[...] + x_ref[...])


def _pdd2(adj, x, scale, shift, w1, b1, w2, b2, w3, b3):
    return pl.pallas_call(
        _pdd2_body,
        grid=(NNB,),
        in_specs=[
            pl.BlockSpec((BN, 256), lambda i: (i, 0)),
            pl.BlockSpec((BN, 256), lambda i: (i, 0)),
            pl.BlockSpec((1, 256), lambda i: (0, 0)),
            pl.BlockSpec((1, 256), lambda i: (0, 0)),
            pl.BlockSpec((256, 512), lambda i: (0, 0)),
            pl.BlockSpec((1, 512), lambda i: (0, 0)),
            pl.BlockSpec((256, 256), lambda i: (0, 0)),
            pl.BlockSpec((1, 256), lambda i: (0, 0)),
            pl.BlockSpec((256, 256), lambda i: (0, 0)),
            pl.BlockSpec((1, 256), lambda i: (0, 0)),
        ],
        out_specs=pl.BlockSpec((BN, 256), lambda i: (i, 0)),
        out_shape=jax.ShapeDtypeStruct((N, 256), _f32),
        compiler_params=pltpu.CompilerParams(dimension_semantics=("parallel",)),
    )(adj, x, scale, shift, w1, b1, w2, b2, w3, b3)


def _pool_body(num_ref, den_ref, xs_ref, b_ref, sums_ref, cnt_ref):
    i = pl.program_id(0)
    den = jnp.max(den_ref[...], axis=1, keepdims=True)
    x3 = num_ref[...] * (1.0 / (den + 1e-16)) + xs_ref[...]
    b = jnp.max(b_ref[...], axis=1, keepdims=True)
    ids = lax.broadcasted_iota(jnp.int32, (1, 128), 1).astype(_f32)
    onehot = (b == ids).astype(_f32)
    part = lax.dot_general(onehot, x3, (((0,), (0,)), ((), ())),
                           preferred_element_type=_f32)
    cpart = jnp.sum(onehot, axis=0, keepdims=True)

    @pl.when(i == 0)
    def _():
        sums_ref[...] = jnp.zeros_like(sums_ref)
        cnt_ref[...] = jnp.zeros_like(cnt_ref)

    sums_ref[...] += part
    cnt_ref[...] += jnp.broadcast_to(cpart, (8, 128))


def _pool(num, den, xs, batchf):
    return pl.pallas_call(
        _pool_body,
        grid=(NNB,),
        in_specs=[
            pl.BlockSpec((BN, 256), lambda i: (i, 0)),
            pl.BlockSpec((BN, 16), lambda i: (i, 0)),
            pl.BlockSpec((BN, 256), lambda i: (i, 0)),
            pl.BlockSpec((BN, 16), lambda i: (i, 0)),
        ],
        out_specs=[
            pl.BlockSpec((128, 256), lambda i: (0, 0)),
            pl.BlockSpec((8, 128), lambda i: (0, 0)),
        ],
        out_shape=[
            jax.ShapeDtypeStruct((128, 256), _f32),
            jax.ShapeDtypeStruct((8, 128), _f32),
        ],
        compiler_params=pltpu.CompilerParams(dimension_semantics=("arbitrary",)),
    )(num, den, xs, batchf)


def _head_body(p_ref, wf_ref, bf_ref, wo_ref, bo_ref, o_ref):
    pooled = p_ref[...]
    f = pooled + _silu(jnp.dot(pooled, wf_ref[...], preferred_element_type=_f32)
                       + bf_ref[...])
    o_ref[...] = jnp.dot(f, wo_ref[...], preferred_element_type=_f32) + bo_ref[...]


def _head(pooled, wf, bf, wop, bop):
    return pl.pallas_call(
        _head_body,
        in_specs=[
            pl.BlockSpec((64, 256), lambda: (0, 0)),
            pl.BlockSpec((256, 256), lambda: (0, 0)),
            pl.BlockSpec((1, 256), lambda: (0, 0)),
            pl.BlockSpec((256, 128), lambda: (0, 0)),
            pl.BlockSpec((1, 128), lambda: (0, 0)),
        ],
        out_specs=pl.BlockSpec((64, 128), lambda: (0, 0)),
        out_shape=jax.ShapeDtypeStruct((64, 128), _f32),
    )(pooled, wf, bf, wop, bop)


def _conv_sparse(q, k, v, ee, src, dst):
    qd, ks, vs = _sc_gather_fn()(src, dst, q, k, v)
    alpha, bmax = _alpha(qd, ks, ee)
    cmax = jnp.max(bmax).reshape(1, 1)
    ynum, yex = _y(cmax, alpha, vs, ee)
    yex = yex.reshape(E)
    zn = jnp.zeros((ACCR * 256,), _f32)
    ze = jnp.zeros((ACCR * 16,), _f32)
    onum, oden = _sc_scatter_fn()(ynum, yex, dst, zn, ze)
    num = onum.reshape(NW * NR, 256)[:N]
    den = oden.reshape(NW * NR, 16)[:N]
    return num, den


def kernel(node, edge_attr, pdd, edge_index, batch, Wa1, ba1, Wa2, ba2,
           Wemb, bemb, Wp, bp, Wq, bq, Wk, bk, Wv, bv, Wed, bed, Ws, bs,
           W1p, b1p, W2p, b2p, W3p, b3p, gbn, bbn, Wf, bf, Wo, bo):
    ef = -1.0 / jnp.linalg.norm(edge_attr, axis=1)
    ef3d = ef.reshape(E, 1)
    wedcat = jnp.concatenate([Wed[0], Wed[1], Wed[2]], axis=1)
    bedcat = jnp.concatenate([bed[0], bed[1], bed[2]], axis=0).reshape(1, 768)
    ee0, ee1, ee2 = _embed(ef3d, Wemb, bemb.reshape(1, 256), wedcat, bedcat)
    ees = (ee0, ee1, ee2)

    nodep = jnp.pad(node, ((0, 0), (0, 128 - node.shape[1])))
    pddp = jnp.pad(pdd, ((0, 0), (0, 128 - pdd.shape[1])))
    wa1p = jnp.pad(Wa1, ((0, 128 - Wa1.shape[0]), (0, 0)))
    wpp = jnp.pad(Wp, ((0, 128 - Wp.shape[0]), (0, 0)))
    x, p = _x0p0(nodep, pddp, wa1p, ba1.reshape(1, 256), Wa2,
                 ba2.reshape(1, 256), wpp, bp.reshape(1, 256))

    src = edge_index[0]
    dst = edge_index[1]

    for c in range(3):
        wcat = jnp.concatenate([Wq[c], Wk[c], Wv[c], Ws[c]], axis=1)
        bcat = jnp.concatenate([bq[c], bk[c], bv[c], bs[c]], axis=0).reshape(1, 1024)
        q, k, v, xs = _qkvs(x, wcat, bcat)
        num, den = _conv_sparse(q, k, v, ees[c], src, dst)
        if c < 2:
            xc, adj, s1, s2 = _pdd1(num, den, xs, p)
            mu = jnp.sum(s1, axis=(0, 1)) / N
            var = jnp.sum(s2, axis=(0, 1)) / N - mu * mu
            scale = gbn[c] / jnp.sqrt(var + 1e-5)
            shift = bbn[c] - mu * scale
            x = _pdd2(adj, xc, scale.reshape(1, 256), shift.reshape(1, 256),
                      W1p[c], b1p[c].reshape(1, 512), W2p[c], b2p[c].reshape(1, 256),
                      W3p[c], b3p[c].reshape(1, 256))
            p = adj
        else:
            batchf = jnp.broadcast_to(batch.astype(_f32)[:, None], (N, 16))
            sums, cnt = _pool(num, den, xs, batchf)
            pooled = sums[:G] / jnp.maximum(cnt[0, :G], 1.0)[:, None]
            wop = jnp.pad(Wo, ((0, 0), (0, 127)))
            bop = jnp.pad(bo, ((0, 127))).reshape(1, 128)
            res = _head(pooled, Wf, bf.reshape(1, 256), wop, bop)
            return res[:, 0]


# speedup vs baseline: 2.1774x; 1.1375x over previous
import functools

import jax
import jax.numpy as jnp
from jax import lax
from jax.experimental import pallas as pl
from jax.experimental.pallas import tpu as pltpu
from jax.experimental.pallas import tpu_sc as plsc

N = 10000
E = 160000
G = 64
D = 256

BE = 640
NEB = E // BE
BN = 400
NNB = N // BN

NC = 2
NS = 16
NW = NC * NS
CE = E // NW
GB = 40
NGC = CE // GB
SC_C = 2000
SC_NCH = E // SC_C
NR = 320
DUMP = NR
ACCR = NR + 8
UROW = 64
RCAP = SC_C + 2 * UROW
_f32 = jnp.float32


def _silu(x):
    return x * jax.nn.sigmoid(x)


def _embed_body(ef_ref, wemb_ref, bemb_ref, wed_ref, bed_ref, e0_ref, e1_ref, e2_ref):
    ef = ef_ref[...]
    centers = -6.0 + (6.0 / 255.0) * lax.broadcasted_iota(jnp.int32, (1, 256), 1).astype(_f32)
    gamma = 1.0 / (6.0 / 255.0)
    diff = ef - centers
    rbf = jnp.exp(-gamma * diff * diff)
    z = jnp.dot(rbf, wemb_ref[...], preferred_element_type=_f32) + bemb_ref[...]
    e = _silu(z)
    ee = jnp.dot(e, wed_ref[...], preferred_element_type=_f32) + bed_ref[...]
    e0_ref[...] = ee[:, :256]
    e1_ref[...] = ee[:, 256:512]
    e2_ref[...] = ee[:, 512:]


def _embed(ef3d, wemb, bemb, wedcat, bedcat):
    return pl.pallas_call(
        _embed_body,
        grid=(NEB,),
        in_specs=[
            pl.BlockSpec((BE, 1), lambda i: (i, 0)),
            pl.BlockSpec((256, 256), lambda i: (0, 0)),
            pl.BlockSpec((1, 256), lambda i: (0, 0)),
            pl.BlockSpec((256, 768), lambda i: (0, 0)),
            pl.BlockSpec((1, 768), lambda i: (0, 0)),
        ],
        out_specs=[pl.BlockSpec((BE, 256), lambda i: (i, 0))] * 3,
        out_shape=[jax.ShapeDtypeStruct((E, 256), _f32)] * 3,
        compiler_params=pltpu.CompilerParams(dimension_semantics=("parallel",)),
    )(ef3d, wemb, bemb, wedcat, bedcat)


def _x0p0_body(node_ref, pdd_ref, wa1_ref, ba1_ref, wa2_ref, ba2_ref,
               wp_ref, bp_ref, x0_ref, p0_ref):
    h = jnp.dot(node_ref[...], wa1_ref[...], preferred_element_type=_f32) + ba1_ref[...]
    h = _silu(h)
    x0_ref[...] = jnp.dot(h, wa2_ref[...], preferred_element_type=_f32) + ba2_ref[...]
    p0_ref[...] = jnp.dot(pdd_ref[...], wp_ref[...], preferred_element_type=_f32) + bp_ref[...]


def _x0p0(nodep, pddp, wa1p, ba1, wa2, ba2, wpp, bp):
    return pl.pallas_call(
        _x0p0_body,
        grid=(NNB,),
        in_specs=[
            pl.BlockSpec((BN, 128), lambda i: (i, 0)),
            pl.BlockSpec((BN, 128), lambda i: (i, 0)),
            pl.BlockSpec((128, 256), lambda i: (0, 0)),
            pl.BlockSpec((1, 256), lambda i: (0, 0)),
            pl.BlockSpec((256, 256), lambda i: (0, 0)),
            pl.BlockSpec((1, 256), lambda i: (0, 0)),
            pl.BlockSpec((128, 256), lambda i: (0, 0)),
            pl.BlockSpec((1, 256), lambda i: (0, 0)),
        ],
        out_specs=[pl.BlockSpec((BN, 256), lambda i: (i, 0))] * 2,
        out_shape=[jax.ShapeDtypeStruct((N, 256), _f32)] * 2,
        compiler_params=pltpu.CompilerParams(dimension_semantics=("parallel",)),
    )(nodep, pddp, wa1p, ba1, wa2, ba2, wpp, bp)


def _qkvs_body(x_ref, w_ref, b_ref, q_ref, kv_ref, s_ref):
    o = jnp.dot(x_ref[...], w_ref[...], preferred_element_type=_f32) + b_ref[...]
    q_ref[...] = o[:, :256]
    kv_ref[...] = o[:, 256:768]
    s_ref[...] = o[:, 768:]


def _qkvs(x, wcat, bcat):
    return pl.pallas_call(
        _qkvs_body,
        grid=(NNB,),
        in_specs=[
            pl.BlockSpec((BN, 256), lambda i: (i, 0)),
            pl.BlockSpec((256, 1024), lambda i: (0, 0)),
            pl.BlockSpec((1, 1024), lambda i: (0, 0)),
        ],
        out_specs=[
            pl.BlockSpec((BN, 256), lambda i: (i, 0)),
            pl.BlockSpec((BN, 512), lambda i: (i, 0)),
            pl.BlockSpec((BN, 256), lambda i: (i, 0)),
        ],
        out_shape=[
            jax.ShapeDtypeStruct((N, 256), _f32),
            jax.ShapeDtypeStruct((N, 512), _f32),
            jax.ShapeDtypeStruct((N, 256), _f32),
        ],
        compiler_params=pltpu.CompilerParams(dimension_semantics=("parallel",)),
    )(x, wcat, bcat)


@functools.cache
def _sc_gather_fn():
    mesh = plsc.VectorSubcoreMesh(core_axis_name="c", subcore_axis_name="s")
    return functools.partial(
        pl.kernel,
        mesh=mesh,
        out_type=[
            jax.ShapeDtypeStruct((E, 256), _f32),
            jax.ShapeDtypeStruct((E, 512), _f32),
        ],
        scratch_types=[
            pltpu.VMEM((GB,), jnp.int32),
            pltpu.VMEM((GB,), jnp.int32),
            pltpu.VMEM((GB, 256), _f32),
            pltpu.VMEM((GB, 512), _f32),
            pltpu.SemaphoreType.DMA,
            pltpu.SemaphoreType.DMA,
            pltpu.SemaphoreType.DMA,
            pltpu.SemaphoreType.DMA,
        ],
    )(_sc_gather_body)


def _sc_gather_body(src_hbm, dst_hbm, q_hbm, kv_hbm,
                    qd_hbm, kvj_hbm, srci, dsti, qrow, kvrow,
                    semq, semkv,sq, skv):
    c = lax.axis_index("c")
    s = lax.axis_index("s")
    wid = s * NC + c
    base0 = wid * CE

    def body(j, carry):
        base = base0 + j * GB
        pltpu.sync_copy(src_hbm.at[pl.ds(base, GB)], srci)
        pltpu.sync_copy(dst_hbm.at[pl.ds(base, GB)], dsti)
        gq = pltpu.async_copy(q_hbm.at[dsti], qrow, semq)
        gkv = pltpu.async_copy(kv_hbm.at[srci], kvrow, semkv)
        gq.wait()
        sq_ = pltpu.async_copy(qrow, qd_hbm.at[pl.ds(base, GB)], sq)
        gkv.wait()
        skv_ = pltpu.async_copy(kvrow, kvj_hbm.at[pl.ds(base, GB)], skv)
        sq_.wait()
        skv_.wait()
        return carry

    lax.fori_loop(0, NGC, body, 0)


def _alpha_body(qd_ref, ks_ref, ee_ref, a_ref, m_ref):
    a = jnp.sum(qd_ref[...] * (ks_ref[...] + ee_ref[...]), axis=1, keepdims=True) / 16.0
    a_ref[...] = a
    m_ref[...] = jnp.full((1, 1, 128), jnp.max(a), _f32)


def _alpha(qd, kvj, ee):
    return pl.pallas_call(
        _alpha_body,
        grid=(NEB,),
        in_specs=[
            pl.BlockSpec((BE, 256), lambda i: (i, 0)),
            pl.BlockSpec((BE, 256), lambda i: (i, 0)),
            pl.BlockSpec((BE, 256), lambda i: (i, 0)),
        ],
        out_specs=[
            pl.BlockSpec((BE, 1), lambda i: (i, 0)),
            pl.BlockSpec((1, 1, 128), lambda i: (i, 0, 0)),
        ],
        out_shape=[
            jax.ShapeDtypeStruct((E, 1), _f32),
            jax.ShapeDtypeStruct((NEB, 1, 128), _f32),
        ],
        compiler_params=pltpu.CompilerParams(dimension_semantics=("parallel",)),
    )(qd, kvj, ee)


def _y_body(c_ref, a_ref, vs_ref, ee_ref, yn_ref, ye_ref):
    ex = jnp.exp(a_ref[...] - c_ref[0, 0])
    ye_ref[...] = ex
    yn_ref[...] = (vs_ref[...] + ee_ref[...]) * ex


def _y(cmax, alpha, kvj, ee):
    return pl.pallas_call(
        _y_body,
        grid=(NEB,),
        in_specs=[
            pl.BlockSpec(memory_space=pltpu.SMEM),
            pl.BlockSpec((BE, 1), lambda i: (i, 0)),
            pl.BlockSpec((BE, 256), lambda i: (i, 1)),
            pl.BlockSpec((BE, 256), lambda i: (i, 0)),
        ],
        out_specs=[
            pl.BlockSpec((BE, 256), lambda i: (i, 0)),
            pl.BlockSpec((BE, 1), lambda i: (i, 0)),
        ],
        out_shape=[
            jax.ShapeDtypeStruct((E, 256), _f32),
            jax.ShapeDtypeStruct((E, 1), _f32),
        ],
        compiler_params=pltpu.CompilerParams(dimension_semantics=("parallel",)),
    )(cmax, alpha, kvj, ee)


@functools.cache
def _sc_scatter_fn():
    mesh = plsc.VectorSubcoreMesh(core_axis_name="c", subcore_axis_name="s")
    return functools.partial(
        pl.kernel,
        mesh=mesh,
        compiler_params=pltpu.CompilerParams(needs_layout_passes=False),
        out_type=[
            jax.ShapeDtypeStruct((NW, NR * 256), _f32),
            jax.ShapeDtypeStruct((NW, NR * 16), _f32),
        ],
        scratch_types=[
            pltpu.VMEM((SC_C,), jnp.int32),
            pltpu.VMEM((SC_C,), _f32),
            pltpu.VMEM((RCAP,), jnp.int32),
            pltpu.VMEM((RCAP,), jnp.int32),
            pltpu.VMEM((RCAP,), _f32),
            pltpu.VMEM((UROW, 256), _f32),
            pltpu.VMEM((ACCR * 256,), _f32),
            pltpu.VMEM((ACCR * 16,), _f32),
            pltpu.SemaphoreType.DMA,
        ],
    )(_sc_scatter_body)


def _sc_scatter_body(yn_hbm, ye_hbm, dst_hbm, zn_hbm, ze_hbm, on_hbm, od_hbm,
                     dbuf, xbuf, ebuf, lbuf, cxbuf, yrow, accn, acce, sem):
    c = lax.axis_index("c")
    s = lax.axis_index("s")
    w = s * NC + c
    lo = w * NR
    lane = lax.iota(jnp.int32, 16)

    pltpu.sync_copy(zn_hbm, accn)
    pltpu.sync_copy(ze_hbm, acce)

    def process_units(n_units):
        def unit_body(u, carry2):
            pltpu.async_copy(yn_hbm.at[ebuf.at[pl.ds(u * UROW, UROW)]], yrow, sem).wait()

            def row_body(r, carry3):
                pos = jnp.full((16,), u * UROW + r, jnp.int32)
                ii = plsc.load_gather(lbuf, [pos])
                exv = plsc.load_gather(cxbuf, [pos])
                basea = ii * 256
                for l in range(16):
                    vals = yrow[r, pl.ds(l * 16, 16)]
                    plsc.addupdate_scatter(accn, [basea + (l * 16) + lane], vals)
                plsc.addupdate_scatter(acce, [ii * 16 + lane], exv)
                return carry3

            lax.fori_loop(0, UROW, row_body, 0)
            return carry2

        lax.fori_loop(0, n_units, unit_body, 0)

    def chunk_body(j, f):
        base = j * SC_C
        pltpu.sync_copy(dst_hbm.at[pl.ds(base, SC_C)], dbuf)
        pltpu.sync_copy(ye_hbm.at[pl.ds(base, SC_C)], xbuf)

        def scan_body(g, f2):
            d16 = dbuf[pl.ds(g * 16, 16)]
            loc = d16 - lo
            ok = (loc >= 0) & (loc < NR)
            eid = base + g * 16 + lane
            plsc.store_compressed(ebuf.at[pl.ds(f2, 16)], eid, mask=ok)
            plsc.store_compressed(lbuf.at[pl.ds(f2, 16)], loc, mask=ok)
            plsc.store_compressed(cxbuf.at[pl.ds(f2, 16)], xbuf[pl.ds(g * 16, 16)], mask=ok)
            cnt = plsc.all_reduce_population_count(ok)
            return f2 + cnt[0]

        f = lax.fori_loop(0, SC_C // 16, scan_body, f)
        n_units = f // UROW
        process_units(n_units)
        rem = f - n_units * UROW
        for t in range(UROW // 16):
            ve = ebuf[pl.ds(n_units * UROW + t * 16, 16)]
            vl = lbuf[pl.ds(n_units * UROW + t * 16, 16)]
            vx = cxbuf[pl.ds(n_units * UROW + t * 16, 16)]
            ebuf[pl.ds(t * 16, 16)] = ve
            lbuf[pl.ds(t * 16, 16)] = vl
            cxbuf[pl.ds(t * 16, 16)] = vx
        return rem

    f = lax.fori_loop(0, SC_NCH, chunk_body, 0)

    def pad_body(t, carry):
        ebuf[pl.ds(f + t * 16, 16)] = jnp.zeros((16,), jnp.int32)
        lbuf[pl.ds(f + t * 16, 16)] = jnp.full((16,), DUMP, jnp.int32)
        cxbuf[pl.ds(f + t * 16, 16)] = jnp.zeros((16,), _f32)
        return carry

    lax.fori_loop(0, UROW // 16, pad_body, 0)
    process_units((f + UROW - 1) // UROW)

    pltpu.sync_copy(accn.at[pl.ds(0, NR * 256)], on_hbm.at[w])
    pltpu.sync_copy(acce.at[pl.ds(0, NR * 16)], od_hbm.at[w])


def _pdd1_body(num_ref, den_ref, xs_ref, p_ref, x_ref, adj_ref, s1_ref, s2_ref):
    den = jnp.max(den_ref[...], axis=1, keepdims=True)
    x = num_ref[...] * (1.0 / (den + 1e-16)) + xs_ref[...]
    adj = p_ref[...] + x
    x_ref[...] = x
    adj_ref[...] = adj
    s1_ref[...] = jnp.sum(adj, axis=0, keepdims=True).reshape(1, 1, 256)
    s2_ref[...] = jnp.sum(adj * adj, axis=0, keepdims=True).reshape(1, 1, 256)


def _pdd1(num, den, xs, p):
    return pl.pallas_call(
        _pdd1_body,
        grid=(NNB,),
        in_specs=[
            pl.BlockSpec((BN, 256), lambda i: (i, 0)),
            pl.BlockSpec((BN, 16), lambda i: (i, 0)),
            pl.BlockSpec((BN, 256), lambda i: (i, 0)),
            pl.BlockSpec((BN, 256), lambda i: (i, 0)),
        ],
        out_specs=[
            pl.BlockSpec((BN, 256), lambda i: (i, 0)),
            pl.BlockSpec((BN, 256), lambda i: (i, 0)),
            pl.BlockSpec((1, 1, 256), lambda i: (i, 0, 0)),
            pl.BlockSpec((1, 1, 256), lambda i: (i, 0, 0)),
        ],
        out_shape=[
            jax.ShapeDtypeStruct((N, 256), _f32),
            jax.ShapeDtypeStruct((N, 256), _f32),
            jax.ShapeDtypeStruct((NNB, 1, 256), _f32),
            jax.ShapeDtypeStruct((NNB, 1, 256), _f32),
        ],
        compiler_params=pltpu.CompilerParams(dimension_semantics=("parallel",)),
    )(num, den, xs, p)


def _pdd2_body(adj_ref, x_ref, sc_ref, sh_ref, w1_ref, b1_ref, w2_ref, b2_ref,
               w3_ref, b3_ref, o_ref):
    h = adj_ref[...] * sc_ref[...] + sh_ref[...]
    h2 = jnp.dot(h, w1_ref[...], preferred_element_type=_f32) + b1_ref[...]
    x1 = h2[:, :256]
    x2 = h2[:, 256:]
    x1 = jnp.dot(x1, w2_ref[...], preferred_element_type=_f32) + b2_ref[...]
    x2 = 0.5 * x2 * (1.0 + lax.erf(x2 * 0.7071067811865476))
    o_ref[...] = (jnp.dot(x1 * x2, w3_ref[...], preferred_element_type=_f32)
                  + b3_ref[...] + x_ref[...])


def _pdd2(adj, x, scale, shift, w1, b1, w2, b2, w3, b3):
    return pl.pallas_call(
        _pdd2_body,
        grid=(NNB,),
        in_specs=[
            pl.BlockSpec((BN, 256), lambda i: (i, 0)),
            pl.BlockSpec((BN, 256), lambda i: (i, 0)),
            pl.BlockSpec((1, 256), lambda i: (0, 0)),
            pl.BlockSpec((1, 256), lambda i: (0, 0)),
            pl.BlockSpec((256, 512), lambda i: (0, 0)),
            pl.BlockSpec((1, 512), lambda i: (0, 0)),
            pl.BlockSpec((256, 256), lambda i: (0, 0)),
            pl.BlockSpec((1, 256), lambda i: (0, 0)),
            pl.BlockSpec((256, 256), lambda i: (0, 0)),
            pl.BlockSpec((1, 256), lambda i: (0, 0)),
        ],
        out_specs=pl.BlockSpec((BN, 256), lambda i: (i, 0)),
        out_shape=jax.ShapeDtypeStruct((N, 256), _f32),
        compiler_params=pltpu.CompilerParams(dimension_semantics=("parallel",)),
    )(adj, x, scale, shift, w1, b1, w2, b2, w3, b3)


def _pool_body(num_ref, den_ref, xs_ref, b_ref, sums_ref, cnt_ref):
    i = pl.program_id(0)
    den = jnp.max(den_ref[...], axis=1, keepdims=True)
    x3 = num_ref[...] * (1.0 / (den + 1e-16)) + xs_ref[...]
    b = jnp.max(b_ref[...], axis=1, keepdims=True)
    ids = lax.broadcasted_iota(jnp.int32, (1, 128), 1).astype(_f32)
    onehot = (b == ids).astype(_f32)
    part = lax.dot_general(onehot, x3, (((0,), (0,)), ((), ())),
                           preferred_element_type=_f32)
    cpart = jnp.sum(onehot, axis=0, keepdims=True)

    @pl.when(i == 0)
    def _():
        sums_ref[...] = jnp.zeros_like(sums_ref)
        cnt_ref[...] = jnp.zeros_like(cnt_ref)

    sums_ref[...] += part
    cnt_ref[...] += jnp.broadcast_to(cpart, (8, 128))


def _pool(num, den, xs, batchf):
    return pl.pallas_call(
        _pool_body,
        grid=(NNB,),
        in_specs=[
            pl.BlockSpec((BN, 256), lambda i: (i, 0)),
            pl.BlockSpec((BN, 16), lambda i: (i, 0)),
            pl.BlockSpec((BN, 256), lambda i: (i, 0)),
            pl.BlockSpec((BN, 16), lambda i: (i, 0)),
        ],
        out_specs=[
            pl.BlockSpec((128, 256), lambda i: (0, 0)),
            pl.BlockSpec((8, 128), lambda i: (0, 0)),
        ],
        out_shape=[
            jax.ShapeDtypeStruct((128, 256), _f32),
            jax.ShapeDtypeStruct((8, 128), _f32),
        ],
        compiler_params=pltpu.CompilerParams(dimension_semantics=("arbitrary",)),
    )(num, den, xs, batchf)


def _head_body(p_ref, wf_ref, bf_ref, wo_ref, bo_ref, o_ref):
    pooled = p_ref[...]
    f = pooled + _silu(jnp.dot(pooled, wf_ref[...], preferred_element_type=_f32)
                       + bf_ref[...])
    o_ref[...] = jnp.dot(f, wo_ref[...], preferred_element_type=_f32) + bo_ref[...]


def _head(pooled, wf, bf, wop, bop):
    return pl.pallas_call(
        _head_body,
        in_specs=[
            pl.BlockSpec((64, 256), lambda: (0, 0)),
            pl.BlockSpec((256, 256), lambda: (0, 0)),
            pl.BlockSpec((1, 256), lambda: (0, 0)),
            pl.BlockSpec((256, 128), lambda: (0, 0)),
            pl.BlockSpec((1, 128), lambda: (0, 0)),
        ],
        out_specs=pl.BlockSpec((64, 128), lambda: (0, 0)),
        out_shape=jax.ShapeDtypeStruct((64, 128), _f32),
    )(pooled, wf, bf, wop, bop)


def _conv_sparse(q, kv, ee, src, dst):
    qd, kvj = _sc_gather_fn()(src, dst, q, kv)
    alpha, bmax = _alpha(qd, kvj, ee)
    cmax = jnp.max(bmax).reshape(1, 1)
    ynum, yex = _y(cmax, alpha, kvj, ee)
    yex = yex.reshape(E)
    zn = jnp.zeros((ACCR * 256,), _f32)
    ze = jnp.zeros((ACCR * 16,), _f32)
    onum, oden = _sc_scatter_fn()(ynum, yex, dst, zn, ze)
    num = onum.reshape(NW * NR, 256)[:N]
    den = oden.reshape(NW * NR, 16)[:N]
    return num, den


def kernel(node, edge_attr, pdd, edge_index, batch, Wa1, ba1, Wa2, ba2,
           Wemb, bemb, Wp, bp, Wq, bq, Wk, bk, Wv, bv, Wed, bed, Ws, bs,
           W1p, b1p, W2p, b2p, W3p, b3p, gbn, bbn, Wf, bf, Wo, bo):
    ef = -1.0 / jnp.linalg.norm(edge_attr, axis=1)
    ef3d = ef.reshape(E, 1)
    wedcat = jnp.concatenate([Wed[0], Wed[1], Wed[2]], axis=1)
    bedcat = jnp.concatenate([bed[0], bed[1], bed[2]], axis=0).reshape(1, 768)
    ee0, ee1, ee2 = _embed(ef3d, Wemb, bemb.reshape(1, 256), wedcat, bedcat)
    ees = (ee0, ee1, ee2)

    nodep = jnp.pad(node, ((0, 0), (0, 128 - node.shape[1])))
    pddp = jnp.pad(pdd, ((0, 0), (0, 128 - pdd.shape[1])))
    wa1p = jnp.pad(Wa1, ((0, 128 - Wa1.shape[0]), (0, 0)))
    wpp = jnp.pad(Wp, ((0, 128 - Wp.shape[0]), (0, 0)))
    x, p = _x0p0(nodep, pddp, wa1p, ba1.reshape(1, 256), Wa2,
                 ba2.reshape(1, 256), wpp, bp.reshape(1, 256))

    src = edge_index[0]
    dst = edge_index[1]

    for c in range(3):
        wcat = jnp.concatenate([Wq[c], Wk[c], Wv[c], Ws[c]], axis=1)
        bcat = jnp.concatenate([bq[c], bk[c], bv[c], bs[c]], axis=0).reshape(1, 1024)
        q, kv, xs = _qkvs(x, wcat, bcat)
        num, den = _conv_sparse(q, kv, ees[c], src, dst)
        if c < 2:
            xc, adj, s1, s2 = _pdd1(num, den, xs, p)
            mu = jnp.sum(s1, axis=(0, 1)) / N
            var = jnp.sum(s2, axis=(0, 1)) / N - mu * mu
            scale = gbn[c] / jnp.sqrt(var + 1e-5)
            shift = bbn[c] - mu * scale
            x = _pdd2(adj, xc, scale.reshape(1, 256), shift.reshape(1, 256),
                      W1p[c], b1p[c].reshape(1, 512), W2p[c], b2p[c].reshape(1, 256),
                      W3p[c], b3p[c].reshape(1, 256))
            p = adj
        else:
            batchf = jnp.broadcast_to(batch.astype(_f32)[:, None], (N, 16))
            sums, cnt = _pool(num, den, xs, batchf)
            pooled = sums[:G] / jnp.maximum(cnt[0, :G], 1.0)[:, None]
            wop = jnp.pad(Wo, ((0, 0), (0, 127)))
            bop = jnp.pad(bo, ((0, 127))).reshape(1, 128)
            res = _head(pooled, Wf, bf.reshape(1, 256), wop, bop)
            return res[:, 0]


# speedup vs baseline: 2.2767x; 1.0456x over previous
import functools

import jax
import jax.numpy as jnp
from jax import lax
from jax.experimental import pallas as pl
from jax.experimental.pallas import tpu as pltpu
from jax.experimental.pallas import tpu_sc as plsc

N = 10000
E = 160000
G = 64
D = 256

BE = 640
NEB = E // BE
BN = 400
NNB = N // BN

NC = 2
NS = 16
NW = NC * NS
CE = E // NW
GB = 40
NGC = CE // GB
SC_C = 2000
SC_NCH = E // SC_C
NR = 320
DUMP = NR
ACCR = NR + 1
UROW = 64
RCAP = SC_C + 2 * UROW
_f32 = jnp.float32


def _silu(x):
    return x * jax.nn.sigmoid(x)


def _embed_body(ef_ref, wemb_ref, bemb_ref, wed_ref, bed_ref, e0_ref, e1_ref, e2_ref):
    ef = ef_ref[...]
    centers = -6.0 + (6.0 / 255.0) * lax.broadcasted_iota(jnp.int32, (1, 256), 1).astype(_f32)
    gamma = 1.0 / (6.0 / 255.0)
    diff = ef - centers
    rbf = jnp.exp(-gamma * diff * diff)
    z = jnp.dot(rbf, wemb_ref[...], preferred_element_type=_f32) + bemb_ref[...]
    e = _silu(z)
    ee = jnp.dot(e, wed_ref[...], preferred_element_type=_f32) + bed_ref[...]
    e0_ref[...] = ee[:, :256]
    e1_ref[...] = ee[:, 256:512]
    e2_ref[...] = ee[:, 512:]


def _embed(ef3d, wemb, bemb, wedcat, bedcat):
    return pl.pallas_call(
        _embed_body,
        grid=(NEB,),
        in_specs=[
            pl.BlockSpec((BE, 1), lambda i: (i, 0)),
            pl.BlockSpec((256, 256), lambda i: (0, 0)),
            pl.BlockSpec((1, 256), lambda i: (0, 0)),
            pl.BlockSpec((256, 768), lambda i: (0, 0)),
            pl.BlockSpec((1, 768), lambda i: (0, 0)),
        ],
        out_specs=[pl.BlockSpec((BE, 256), lambda i: (i, 0))] * 3,
        out_shape=[jax.ShapeDtypeStruct((E, 256), _f32)] * 3,
        compiler_params=pltpu.CompilerParams(dimension_semantics=("parallel",)),
    )(ef3d, wemb, bemb, wedcat, bedcat)


def _x0p0_body(node_ref, pdd_ref, wa1_ref, ba1_ref, wa2_ref, ba2_ref,
               wp_ref, bp_ref, x0_ref, p0_ref):
    h = jnp.dot(node_ref[...], wa1_ref[...], preferred_element_type=_f32) + ba1_ref[...]
    h = _silu(h)
    x0_ref[...] = jnp.dot(h, wa2_ref[...], preferred_element_type=_f32) + ba2_ref[...]
    p0_ref[...] = jnp.dot(pdd_ref[...], wp_ref[...], preferred_element_type=_f32) + bp_ref[...]


def _x0p0(nodep, pddp, wa1p, ba1, wa2, ba2, wpp, bp):
    return pl.pallas_call(
        _x0p0_body,
        grid=(NNB,),
        in_specs=[
            pl.BlockSpec((BN, 128), lambda i: (i, 0)),
            pl.BlockSpec((BN, 128), lambda i: (i, 0)),
            pl.BlockSpec((128, 256), lambda i: (0, 0)),
            pl.BlockSpec((1, 256), lambda i: (0, 0)),
            pl.BlockSpec((256, 256), lambda i: (0, 0)),
            pl.BlockSpec((1, 256), lambda i: (0, 0)),
            pl.BlockSpec((128, 256), lambda i: (0, 0)),
            pl.BlockSpec((1, 256), lambda i: (0, 0)),
        ],
        out_specs=[pl.BlockSpec((BN, 256), lambda i: (i, 0))] * 2,
        out_shape=[jax.ShapeDtypeStruct((N, 256), _f32)] * 2,
        compiler_params=pltpu.CompilerParams(dimension_semantics=("parallel",)),
    )(nodep, pddp, wa1p, ba1, wa2, ba2, wpp, bp)


def _qkvs_body(x_ref, w_ref, b_ref, q_ref, kv_ref, s_ref):
    o = jnp.dot(x_ref[...], w_ref[...], preferred_element_type=_f32) + b_ref[...]
    q_ref[...] = o[:, :256]
    kv_ref[...] = o[:, 256:768]
    s_ref[...] = o[:, 768:]


def _qkvs(x, wcat, bcat):
    return pl.pallas_call(
        _qkvs_body,
        grid=(NNB,),
        in_specs=[
            pl.BlockSpec((BN, 256), lambda i: (i, 0)),
            pl.BlockSpec((256, 1024), lambda i: (0, 0)),
            pl.BlockSpec((1, 1024), lambda i: (0, 0)),
        ],
        out_specs=[
            pl.BlockSpec((BN, 256), lambda i: (i, 0)),
            pl.BlockSpec((BN, 512), lambda i: (i, 0)),
            pl.BlockSpec((BN, 256), lambda i: (i, 0)),
        ],
        out_shape=[
            jax.ShapeDtypeStruct((N, 256), _f32),
            jax.ShapeDtypeStruct((N, 512), _f32),
            jax.ShapeDtypeStruct((N, 256), _f32),
        ],
        compiler_params=pltpu.CompilerParams(dimension_semantics=("parallel",)),
    )(x, wcat, bcat)


@functools.cache
def _sc_gather_fn():
    mesh = plsc.VectorSubcoreMesh(core_axis_name="c", subcore_axis_name="s")
    return functools.partial(
        pl.kernel,
        mesh=mesh,
        out_type=[
            jax.ShapeDtypeStruct((E, 256), _f32),
            jax.ShapeDtypeStruct((E, 512), _f32),
        ],
        scratch_types=[
            pltpu.VMEM((2 * GB,), jnp.int32),
            pltpu.VMEM((2 * GB,), jnp.int32),
            pltpu.VMEM((2 * GB, 256), _f32),
            pltpu.VMEM((2 * GB, 512), _f32),
            pltpu.SemaphoreType.DMA((2,)),
            pltpu.SemaphoreType.DMA((2,)),
            pltpu.SemaphoreType.DMA((2,)),
            pltpu.SemaphoreType.DMA((2,)),
        ],
    )(_sc_gather_body)


def _sc_gather_body(src_hbm, dst_hbm, q_hbm, kv_hbm,
                    qd_hbm, kvj_hbm, srci, dsti, qrow, kvrow,
                    gq, gkv, sq, skv):
    c = lax.axis_index("c")
    s = lax.axis_index("s")
    wid = s * NC + c
    base0 = wid * CE

    def load_and_gather(j, h):
        base = base0 + j * GB
        pltpu.sync_copy(src_hbm.at[pl.ds(base, GB)], srci.at[pl.ds(h * GB, GB)])
        pltpu.sync_copy(dst_hbm.at[pl.ds(base, GB)], dsti.at[pl.ds(h * GB, GB)])
        pltpu.async_copy(q_hbm.at[dsti.at[pl.ds(h * GB, GB)]],
                         qrow.at[pl.ds(h * GB, GB)], gq.at[h])
        pltpu.async_copy(kv_hbm.at[srci.at[pl.ds(h * GB, GB)]],
                         kvrow.at[pl.ds(h * GB, GB)], gkv.at[h])

    load_and_gather(0, 0)

    def body(j, carry):
        par = j % 2
        nxt = (j + 1) % 2

        @pl.when(j + 1 < NGC)
        def _():
            @pl.when(j >= 1)
            def _():
                pltpu.make_async_copy(qrow.at[pl.ds(nxt * GB, GB)],
                                      qd_hbm.at[pl.ds(0, GB)], sq.at[nxt]).wait()
                pltpu.make_async_copy(kvrow.at[pl.ds(nxt * GB, GB)],
                                      kvj_hbm.at[pl.ds(0, GB)], skv.at[nxt]).wait()

            load_and_gather(j + 1, nxt)

        base = base0 + j * GB
        pltpu.make_async_copy(q_hbm.at[dsti.at[pl.ds(par * GB, GB)]],
                              qrow.at[pl.ds(par * GB, GB)], gq.at[par]).wait()
        pltpu.make_async_copy(kv_hbm.at[srci.at[pl.ds(par * GB, GB)]],
                              kvrow.at[pl.ds(par * GB, GB)], gkv.at[par]).wait()
        pltpu.async_copy(qrow.at[pl.ds(par * GB, GB)],
                         qd_hbm.at[pl.ds(base, GB)], sq.at[par])
        pltpu.async_copy(kvrow.at[pl.ds(par * GB, GB)],
                         kvj_hbm.at[pl.ds(base, GB)], skv.at[par])
        return carry

    lax.fori_loop(0, NGC, body, 0)

    for h in (0, 1):
        pltpu.make_async_copy(qrow.at[pl.ds(h * GB, GB)],
                              qd_hbm.at[pl.ds(0, GB)], sq.at[h]).wait()
        pltpu.make_async_copy(kvrow.at[pl.ds(h * GB, GB)],
                              kvj_hbm.at[pl.ds(0, GB)], skv.at[h]).wait()


def _alpha_body(qd_ref, ks_ref, ee_ref, a_ref, m_ref):
    a = jnp.sum(qd_ref[...] * (ks_ref[...] + ee_ref[...]), axis=1, keepdims=True) / 16.0
    a_ref[...] = a
    m_ref[...] = jnp.full((1, 1, 128), jnp.max(a), _f32)


def _alpha(qd, kvj, ee):
    return pl.pallas_call(
        _alpha_body,
        grid=(NEB,),
        in_specs=[
            pl.BlockSpec((BE, 256), lambda i: (i, 0)),
            pl.BlockSpec((BE, 256), lambda i: (i, 0)),
            pl.BlockSpec((BE, 256), lambda i: (i, 0)),
        ],
        out_specs=[
            pl.BlockSpec((BE, 1), lambda i: (i, 0)),
            pl.BlockSpec((1, 1, 128), lambda i: (i, 0, 0)),
        ],
        out_shape=[
            jax.ShapeDtypeStruct((E, 1), _f32),
            jax.ShapeDtypeStruct((NEB, 1, 128), _f32),
        ],
        compiler_params=pltpu.CompilerParams(dimension_semantics=("parallel",)),
    )(qd, kvj, ee)


def _y_body(c_ref, a_ref, vs_ref, ee_ref, yn_ref, ye_ref):
    ex = jnp.exp(a_ref[...] - c_ref[0, 0])
    ye_ref[...] = ex
    yn_ref[...] = (vs_ref[...] + ee_ref[...]) * ex


def _y(cmax, alpha, kvj, ee):
    return pl.pallas_call(
        _y_body,
        grid=(NEB,),
        in_specs=[
            pl.BlockSpec(memory_space=pltpu.SMEM),
            pl.BlockSpec((BE, 1), lambda i: (i, 0)),
            pl.BlockSpec((BE, 256), lambda i: (i, 1)),
            pl.BlockSpec((BE, 256), lambda i: (i, 0)),
        ],
        out_specs=[
            pl.BlockSpec((BE, 256), lambda i: (i, 0)),
            pl.BlockSpec((BE, 1), lambda i: (i, 0)),
        ],
        out_shape=[
            jax.ShapeDtypeStruct((E, 256), _f32),
            jax.ShapeDtypeStruct((E, 1), _f32),
        ],
        compiler_params=pltpu.CompilerParams(dimension_semantics=("parallel",)),
    )(cmax, alpha, kvj, ee)


@functools.cache
def _sc_scatter_fn():
    mesh = plsc.VectorSubcoreMesh(core_axis_name="c", subcore_axis_name="s")
    return functools.partial(
        pl.kernel,
        mesh=mesh,
        compiler_params=pltpu.CompilerParams(needs_layout_passes=False),
        out_type=[
            jax.ShapeDtypeStruct((NW, NR * 256), _f32),
            jax.ShapeDtypeStruct((NW, NR * 16), _f32),
        ],
        scratch_types=[
            pltpu.VMEM((SC_C,), jnp.int32),
            pltpu.VMEM((SC_C,), _f32),
            pltpu.VMEM((RCAP,), jnp.int32),
            pltpu.VMEM((RCAP,), jnp.int32),
            pltpu.VMEM((RCAP,), _f32),
            pltpu.VMEM((2 * UROW, 256), _f32),
            pltpu.VMEM((ACCR * 256,), _f32),
            pltpu.VMEM((ACCR * 16,), _f32),
            pltpu.SemaphoreType.DMA((2,)),
        ],
    )(_sc_scatter_body)


def _sc_scatter_body(yn_hbm, ye_hbm, dst_hbm, zn_hbm, ze_hbm, on_hbm, od_hbm,
                     dbuf, xbuf, ebuf, lbuf, cxbuf, yrow, accn, acce, sem):
    c = lax.axis_index("c")
    s = lax.axis_index("s")
    w = s * NC + c
    lo = w * NR
    lane = lax.iota(jnp.int32, 16)

    pltpu.sync_copy(zn_hbm, accn)
    pltpu.sync_copy(ze_hbm, acce)

    def issue_unit(u, h):
        pltpu.async_copy(yn_hbm.at[ebuf.at[pl.ds(u * UROW, UROW)]],
                         yrow.at[pl.ds(h * UROW, UROW)], sem.at[h])

    def process_units(n_units):
        @pl.when(n_units > 0)
        def _():
            issue_unit(0, 0)

        def unit_body(u, carry2):
            par = u % 2

            @pl.when(u + 1 < n_units)
            def _():
                issue_unit(u + 1, (u + 1) % 2)

            pltpu.make_async_copy(
                yn_hbm.at[ebuf.at[pl.ds(u * UROW, UROW)]],
                yrow.at[pl.ds(par * UROW, UROW)], sem.at[par]).wait()

            def row_body(r, carry3):
                pos = jnp.full((16,), u * UROW + r, jnp.int32)
                ii = plsc.load_gather(lbuf, [pos])
                exv = plsc.load_gather(cxbuf, [pos])
                basea = ii * 256
                for l in range(16):
                    vals = yrow[par * UROW + r, pl.ds(l * 16, 16)]
                    plsc.addupdate_scatter(accn, [basea + (l * 16) + lane], vals)
                plsc.addupdate_scatter(acce, [ii * 16 + lane], exv)
                return carry3

            lax.fori_loop(0, UROW, row_body, 0)
            return carry2

        lax.fori_loop(0, n_units, unit_body, 0)

    def chunk_body(j, f):
        base = j * SC_C
        pltpu.sync_copy(dst_hbm.at[pl.ds(base, SC_C)], dbuf)
        pltpu.sync_copy(ye_hbm.at[pl.ds(base, SC_C)], xbuf)

        def scan_body(g, f2):
            d16 = dbuf[pl.ds(g * 16, 16)]
            loc = d16 - lo
            ok = (loc >= 0) & (loc < NR)
            eid = base + g * 16 + lane
            plsc.store_compressed(ebuf.at[pl.ds(f2, 16)], eid, mask=ok)
            plsc.store_compressed(lbuf.at[pl.ds(f2, 16)], loc, mask=ok)
            plsc.store_compressed(cxbuf.at[pl.ds(f2, 16)], xbuf[pl.ds(g * 16, 16)], mask=ok)
            cnt = plsc.all_reduce_population_count(ok)
            return f2 + cnt[0]

        f = lax.fori_loop(0, SC_C // 16, scan_body, f)
        n_units = f // UROW
        process_units(n_units)
        rem = f - n_units * UROW
        for t in range(UROW // 16):
            ve = ebuf[pl.ds(n_units * UROW + t * 16, 16)]
            vl = lbuf[pl.ds(n_units * UROW + t * 16, 16)]
            vx = cxbuf[pl.ds(n_units * UROW + t * 16, 16)]
            ebuf[pl.ds(t * 16, 16)] = ve
            lbuf[pl.ds(t * 16, 16)] = vl
            cxbuf[pl.ds(t * 16, 16)] = vx
        return rem

    f = lax.fori_loop(0, SC_NCH, chunk_body, 0)

    def pad_body(t, carry):
        ebuf[pl.ds(f + t * 16, 16)] = jnp.zeros((16,), jnp.int32)
        lbuf[pl.ds(f + t * 16, 16)] = jnp.full((16,), DUMP, jnp.int32)
        cxbuf[pl.ds(f + t * 16, 16)] = jnp.zeros((16,), _f32)
        return carry

    lax.fori_loop(0, UROW // 16, pad_body, 0)
    process_units((f + UROW - 1) // UROW)

    pltpu.sync_copy(accn.at[pl.ds(0, NR * 256)], on_hbm.at[w])
    pltpu.sync_copy(acce.at[pl.ds(0, NR * 16)], od_hbm.at[w])


def _pdd1_body(num_ref, den_ref, xs_ref, p_ref, x_ref, adj_ref, s1_ref, s2_ref):
    den = jnp.max(den_ref[...], axis=1, keepdims=True)
    x = num_ref[...] * (1.0 / (den + 1e-16)) + xs_ref[...]
    adj = p_ref[...] + x
    x_ref[...] = x
    adj_ref[...] = adj
    s1_ref[...] = jnp.sum(adj, axis=0, keepdims=True).reshape(1, 1, 256)
    s2_ref[...] = jnp.sum(adj * adj, axis=0, keepdims=True).reshape(1, 1, 256)


def _pdd1(num, den, xs, p):
    return pl.pallas_call(
        _pdd1_body,
        grid=(NNB,),
        in_specs=[
            pl.BlockSpec((BN, 256), lambda i: (i, 0)),
            pl.BlockSpec((BN, 16), lambda i: (i, 0)),
            pl.BlockSpec((BN, 256), lambda i: (i, 0)),
            pl.BlockSpec((BN, 256), lambda i: (i, 0)),
        ],
        out_specs=[
            pl.BlockSpec((BN, 256), lambda i: (i, 0)),
            pl.BlockSpec((BN, 256), lambda i: (i, 0)),
            pl.BlockSpec((1, 1, 256), lambda i: (i, 0, 0)),
            pl.BlockSpec((1, 1, 256), lambda i: (i, 0, 0)),
        ],
        out_shape=[
            jax.ShapeDtypeStruct((N, 256), _f32),
            jax.ShapeDtypeStruct((N, 256), _f32),
            jax.ShapeDtypeStruct((NNB, 1, 256), _f32),
            jax.ShapeDtypeStruct((NNB, 1, 256), _f32),
        ],
        compiler_params=pltpu.CompilerParams(dimension_semantics=("parallel",)),
    )(num, den, xs, p)


def _pdd2_body(adj_ref, x_ref, sc_ref, sh_ref, w1_ref, b1_ref, w2_ref, b2_ref,
               w3_ref, b3_ref, o_ref):
    h = adj_ref[...] * sc_ref[...] + sh_ref[...]
    h2 = jnp.dot(h, w1_ref[...], preferred_element_type=_f32) + b1_ref[...]
    x1 = h2[:, :256]
    x2 = h2[:, 256:]
    x1 = jnp.dot(x1, w2_ref[...], preferred_element_type=_f32) + b2_ref[...]
    x2 = 0.5 * x2 * (1.0 + lax.erf(x2 * 0.7071067811865476))
    o_ref[...] = (jnp.dot(x1 * x2, w3_ref[...], preferred_element_type=_f32)
                  + b3_ref[...] + x_ref[...])


def _pdd2(adj, x, scale, shift, w1, b1, w2, b2, w3, b3):
    return pl.pallas_call(
        _pdd2_body,
        grid=(NNB,),
        in_specs=[
            pl.BlockSpec((BN, 256), lambda i: (i, 0)),
            pl.BlockSpec((BN, 256), lambda i: (i, 0)),
            pl.BlockSpec((1, 256), lambda i: (0, 0)),
            pl.BlockSpec((1, 256), lambda i: (0, 0)),
            pl.BlockSpec((256, 512), lambda i: (0, 0)),
            pl.BlockSpec((1, 512), lambda i: (0, 0)),
            pl.BlockSpec((256, 256), lambda i: (0, 0)),
            pl.BlockSpec((1, 256), lambda i: (0, 0)),
            pl.BlockSpec((256, 256), lambda i: (0, 0)),
            pl.BlockSpec((1, 256), lambda i: (0, 0)),
        ],
        out_specs=pl.BlockSpec((BN, 256), lambda i: (i, 0)),
        out_shape=jax.ShapeDtypeStruct((N, 256), _f32),
        compiler_params=pltpu.CompilerParams(dimension_semantics=("parallel",)),
    )(adj, x, scale, shift, w1, b1, w2, b2, w3, b3)


def _pool_body(num_ref, den_ref, xs_ref, b_ref, sums_ref, cnt_ref):
    i = pl.program_id(0)
    den = jnp.max(den_ref[...], axis=1, keepdims=True)
    x3 = num_ref[...] * (1.0 / (den + 1e-16)) + xs_ref[...]
    b = jnp.max(b_ref[...], axis=1, keepdims=True)
    ids = lax.broadcasted_iota(jnp.int32, (1, 128), 1).astype(_f32)
    onehot = (b == ids).astype(_f32)
    part = lax.dot_general(onehot, x3, (((0,), (0,)), ((), ())),
                           preferred_element_type=_f32)
    cpart = jnp.sum(onehot, axis=0, keepdims=True)

    @pl.when(i == 0)
    def _():
        sums_ref[...] = jnp.zeros_like(sums_ref)
        cnt_ref[...] = jnp.zeros_like(cnt_ref)

    sums_ref[...] += part
    cnt_ref[...] += jnp.broadcast_to(cpart, (8, 128))


def _pool(num, den, xs, batchf):
    return pl.pallas_call(
        _pool_body,
        grid=(NNB,),
        in_specs=[
            pl.BlockSpec((BN, 256), lambda i: (i, 0)),
            pl.BlockSpec((BN, 16), lambda i: (i, 0)),
            pl.BlockSpec((BN, 256), lambda i: (i, 0)),
            pl.BlockSpec((BN, 16), lambda i: (i, 0)),
        ],
        out_specs=[
            pl.BlockSpec((128, 256), lambda i: (0, 0)),
            pl.BlockSpec((8, 128), lambda i: (0, 0)),
        ],
        out_shape=[
            jax.ShapeDtypeStruct((128, 256), _f32),
            jax.ShapeDtypeStruct((8, 128), _f32),
        ],
        compiler_params=pltpu.CompilerParams(dimension_semantics=("arbitrary",)),
    )(num, den, xs, batchf)


def _head_body(p_ref, wf_ref, bf_ref, wo_ref, bo_ref, o_ref):
    pooled = p_ref[...]
    f = pooled + _silu(jnp.dot(pooled, wf_ref[...], preferred_element_type=_f32)
                       + bf_ref[...])
    o_ref[...] = jnp.dot(f, wo_ref[...], preferred_element_type=_f32) + bo_ref[...]


def _head(pooled, wf, bf, wop, bop):
    return pl.pallas_call(
        _head_body,
        in_specs=[
            pl.BlockSpec((64, 256), lambda: (0, 0)),
            pl.BlockSpec((256, 256), lambda: (0, 0)),
            pl.BlockSpec((1, 256), lambda: (0, 0)),
            pl.BlockSpec((256, 128), lambda: (0, 0)),
            pl.BlockSpec((1, 128), lambda: (0, 0)),
        ],
        out_specs=pl.BlockSpec((64, 128), lambda: (0, 0)),
        out_shape=jax.ShapeDtypeStruct((64, 128), _f32),
    )(pooled, wf, bf, wop, bop)


def _conv_sparse(q, kv, ee, src, dst):
    qd, kvj = _sc_gather_fn()(src, dst, q, kv)
    alpha, bmax = _alpha(qd, kvj, ee)
    cmax = jnp.max(bmax).reshape(1, 1)
    ynum, yex = _y(cmax, alpha, kvj, ee)
    yex = yex.reshape(E)
    zn = jnp.zeros((ACCR * 256,), _f32)
    ze = jnp.zeros((ACCR * 16,), _f32)
    onum, oden = _sc_scatter_fn()(ynum, yex, dst, zn, ze)
    num = onum.reshape(NW * NR, 256)[:N]
    den = oden.reshape(NW * NR, 16)[:N]
    return num, den


def kernel(node, edge_attr, pdd, edge_index, batch, Wa1, ba1, Wa2, ba2,
           Wemb, bemb, Wp, bp, Wq, bq, Wk, bk, Wv, bv, Wed, bed, Ws, bs,
           W1p, b1p, W2p, b2p, W3p, b3p, gbn, bbn, Wf, bf, Wo, bo):
    ef = -1.0 / jnp.linalg.norm(edge_attr, axis=1)
    ef3d = ef.reshape(E, 1)
    wedcat = jnp.concatenate([Wed[0], Wed[1], Wed[2]], axis=1)
    bedcat = jnp.concatenate([bed[0], bed[1], bed[2]], axis=0).reshape(1, 768)
    ee0, ee1, ee2 = _embed(ef3d, Wemb, bemb.reshape(1, 256), wedcat, bedcat)
    ees = (ee0, ee1, ee2)

    nodep = jnp.pad(node, ((0, 0), (0, 128 - node.shape[1])))
    pddp = jnp.pad(pdd, ((0, 0), (0, 128 - pdd.shape[1])))
    wa1p = jnp.pad(Wa1, ((0, 128 - Wa1.shape[0]), (0, 0)))
    wpp = jnp.pad(Wp, ((0, 128 - Wp.shape[0]), (0, 0)))
    x, p = _x0p0(nodep, pddp, wa1p, ba1.reshape(1, 256), Wa2,
                 ba2.reshape(1, 256), wpp, bp.reshape(1, 256))

    src = edge_index[0]
    dst = edge_index[1]

    for c in range(3):
        wcat = jnp.concatenate([Wq[c], Wk[c], Wv[c], Ws[c]], axis=1)
        bcat = jnp.concatenate([bq[c], bk[c], bv[c], bs[c]], axis=0).reshape(1, 1024)
        q, kv, xs = _qkvs(x, wcat, bcat)
        num, den = _conv_sparse(q, kv, ees[c], src, dst)
        if c < 2:
            xc, adj, s1, s2 = _pdd1(num, den, xs, p)
            mu = jnp.sum(s1, axis=(0, 1)) / N
            var = jnp.sum(s2, axis=(0, 1)) / N - mu * mu
            scale = gbn[c] / jnp.sqrt(var + 1e-5)
            shift = bbn[c] - mu * scale
            x = _pdd2(adj, xc, scale.reshape(1, 256), shift.reshape(1, 256),
                      W1p[c], b1p[c].reshape(1, 512), W2p[c], b2p[c].reshape(1, 256),
                      W3p[c], b3p[c].reshape(1, 256))
            p = adj
        else:
            batchf = jnp.broadcast_to(batch.astype(_f32)[:, None], (N, 16))
            sums, cnt = _pool(num, den, xs, batchf)
            pooled = sums[:G] / jnp.maximum(cnt[0, :G], 1.0)[:, None]
            wop = jnp.pad(Wo, ((0, 0), (0, 127)))
            bop = jnp.pad(bo, ((0, 127))).reshape(1, 128)
            res = _head(pooled, Wf, bf.reshape(1, 256), wop, bop)
            return res[:, 0]


# speedup vs baseline: 2.3331x; 1.0248x over previous
import functools

import jax
import jax.numpy as jnp
from jax import lax
from jax.experimental import pallas as pl
from jax.experimental.pallas import tpu as pltpu
from jax.experimental.pallas import tpu_sc as plsc

N = 10000
E = 160000
G = 64
D = 256

BE = 640
NEB = E // BE
BN = 400
NNB = N // BN

NC = 2
NS = 16
NW = NC * NS
CE = E // NW
GB = 40
NGC = CE // GB
SC_C = 2000
SC_NCH = E // SC_C
NR = 320
DUMP = NR
ACCR = NR + 1
UROW = 64
RCAP = SC_C + 2 * UROW
_f32 = jnp.float32


def _silu(x):
    return x * jax.nn.sigmoid(x)


def _embed_body(ef_ref, wemb_ref, bemb_ref, wed_ref, bed_ref, e0_ref, e1_ref, e2_ref):
    ef = ef_ref[...]
    centers = -6.0 + (6.0 / 255.0) * lax.broadcasted_iota(jnp.int32, (1, 256), 1).astype(_f32)
    gamma = 1.0 / (6.0 / 255.0)
    diff = ef - centers
    rbf = jnp.exp(-gamma * diff * diff)
    z = jnp.dot(rbf, wemb_ref[...], preferred_element_type=_f32) + bemb_ref[...]
    e = _silu(z)
    ee = jnp.dot(e, wed_ref[...], preferred_element_type=_f32) + bed_ref[...]
    e0_ref[...] = ee[:, :256]
    e1_ref[...] = ee[:, 256:512]
    e2_ref[...] = ee[:, 512:]


def _embed(ef3d, wemb, bemb, wedcat, bedcat):
    return pl.pallas_call(
        _embed_body,
        grid=(NEB,),
        in_specs=[
            pl.BlockSpec((BE, 1), lambda i: (i, 0)),
            pl.BlockSpec((256, 256), lambda i: (0, 0)),
            pl.BlockSpec((1, 256), lambda i: (0, 0)),
            pl.BlockSpec((256, 768), lambda i: (0, 0)),
            pl.BlockSpec((1, 768), lambda i: (0, 0)),
        ],
        out_specs=[pl.BlockSpec((BE, 256), lambda i: (i, 0))] * 3,
        out_shape=[jax.ShapeDtypeStruct((E, 256), _f32)] * 3,
        compiler_params=pltpu.CompilerParams(dimension_semantics=("parallel",)),
    )(ef3d, wemb, bemb, wedcat, bedcat)


def _x0p0_body(node_ref, pdd_ref, wa1_ref, ba1_ref, wa2_ref, ba2_ref,
               wp_ref, bp_ref, x0_ref, p0_ref):
    h = jnp.dot(node_ref[...], wa1_ref[...], preferred_element_type=_f32) + ba1_ref[...]
    h = _silu(h)
    x0_ref[...] = jnp.dot(h, wa2_ref[...], preferred_element_type=_f32) + ba2_ref[...]
    p0_ref[...] = jnp.dot(pdd_ref[...], wp_ref[...], preferred_element_type=_f32) + bp_ref[...]


def _x0p0(nodep, pddp, wa1p, ba1, wa2, ba2, wpp, bp):
    return pl.pallas_call(
        _x0p0_body,
        grid=(NNB,),
        in_specs=[
            pl.BlockSpec((BN, 128), lambda i: (i, 0)),
            pl.BlockSpec((BN, 128), lambda i: (i, 0)),
            pl.BlockSpec((128, 256), lambda i: (0, 0)),
            pl.BlockSpec((1, 256), lambda i: (0, 0)),
            pl.BlockSpec((256, 256), lambda i: (0, 0)),
            pl.BlockSpec((1, 256), lambda i: (0, 0)),
            pl.BlockSpec((128, 256), lambda i: (0, 0)),
            pl.BlockSpec((1, 256), lambda i: (0, 0)),
        ],
        out_specs=[pl.BlockSpec((BN, 256), lambda i: (i, 0))] * 2,
        out_shape=[jax.ShapeDtypeStruct((N, 256), _f32)] * 2,
        compiler_params=pltpu.CompilerParams(dimension_semantics=("parallel",)),
    )(nodep, pddp, wa1p, ba1, wa2, ba2, wpp, bp)


def _qkvs_body(x_ref, w_ref, b_ref, q_ref, kv_ref, s_ref):
    o = jnp.dot(x_ref[...], w_ref[...], preferred_element_type=_f32) + b_ref[...]
    q_ref[...] = o[:, :256]
    kv_ref[...] = o[:, 256:768]
    s_ref[...] = o[:, 768:]


def _qkvs(x, wcat, bcat):
    return pl.pallas_call(
        _qkvs_body,
        grid=(NNB,),
        in_specs=[
            pl.BlockSpec((BN, 256), lambda i: (i, 0)),
            pl.BlockSpec((256, 1024), lambda i: (0, 0)),
            pl.BlockSpec((1, 1024), lambda i: (0, 0)),
        ],
        out_specs=[
            pl.BlockSpec((BN, 256), lambda i: (i, 0)),
            pl.BlockSpec((BN, 512), lambda i: (i, 0)),
            pl.BlockSpec((BN, 256), lambda i: (i, 0)),
        ],
        out_shape=[
            jax.ShapeDtypeStruct((N, 256), _f32),
            jax.ShapeDtypeStruct((N, 512), _f32),
            jax.ShapeDtypeStruct((N, 256), _f32),
        ],
        compiler_params=pltpu.CompilerParams(dimension_semantics=("parallel",)),
    )(x, wcat, bcat)


@functools.cache
def _sc_gather_fn():
    mesh = plsc.VectorSubcoreMesh(core_axis_name="c", subcore_axis_name="s")
    return functools.partial(
        pl.kernel,
        mesh=mesh,
        out_type=[
            jax.ShapeDtypeStruct((E, 256), _f32),
            jax.ShapeDtypeStruct((E, 512), _f32),
        ],
        scratch_types=[
            pltpu.VMEM((2 * GB,), jnp.int32),
            pltpu.VMEM((2 * GB,), jnp.int32),
            pltpu.VMEM((2 * GB, 256), _f32),
            pltpu.VMEM((2 * GB, 512), _f32),
            pltpu.SemaphoreType.DMA((2,)),
            pltpu.SemaphoreType.DMA((2,)),
            pltpu.SemaphoreType.DMA((2,)),
            pltpu.SemaphoreType.DMA((2,)),
        ],
    )(_sc_gather_body)


def _sc_gather_body(src_hbm, dst_hbm, q_hbm, kv_hbm,
                    qd_hbm, kvj_hbm, srci, dsti, qrow, kvrow,
                    gq, gkv, sq, skv):
    c = lax.axis_index("c")
    s = lax.axis_index("s")
    wid = s * NC + c
    base0 = wid * CE

    def load_and_gather(j, h):
        base = base0 + j * GB
        pltpu.sync_copy(src_hbm.at[pl.ds(base, GB)], srci.at[pl.ds(h * GB, GB)])
        pltpu.sync_copy(dst_hbm.at[pl.ds(base, GB)], dsti.at[pl.ds(h * GB, GB)])
        pltpu.async_copy(q_hbm.at[dsti.at[pl.ds(h * GB, GB)]],
                         qrow.at[pl.ds(h * GB, GB)], gq.at[h])
        pltpu.async_copy(kv_hbm.at[srci.at[pl.ds(h * GB, GB)]],
                         kvrow.at[pl.ds(h * GB, GB)], gkv.at[h])

    load_and_gather(0, 0)

    def body(j, carry):
        par = j % 2
        nxt = (j + 1) % 2

        @pl.when(j + 1 < NGC)
        def _():
            @pl.when(j >= 1)
            def _():
                pltpu.make_async_copy(qrow.at[pl.ds(nxt * GB, GB)],
                                      qd_hbm.at[pl.ds(0, GB)], sq.at[nxt]).wait()
                pltpu.make_async_copy(kvrow.at[pl.ds(nxt * GB, GB)],
                                      kvj_hbm.at[pl.ds(0, GB)], skv.at[nxt]).wait()

            load_and_gather(j + 1, nxt)

        base = base0 + j * GB
        pltpu.make_async_copy(q_hbm.at[dsti.at[pl.ds(par * GB, GB)]],
                              qrow.at[pl.ds(par * GB, GB)], gq.at[par]).wait()
        pltpu.make_async_copy(kv_hbm.at[srci.at[pl.ds(par * GB, GB)]],
                              kvrow.at[pl.ds(par * GB, GB)], gkv.at[par]).wait()
        pltpu.async_copy(qrow.at[pl.ds(par * GB, GB)],
                         qd_hbm.at[pl.ds(base, GB)], sq.at[par])
        pltpu.async_copy(kvrow.at[pl.ds(par * GB, GB)],
                         kvj_hbm.at[pl.ds(base, GB)], skv.at[par])
        return carry

    lax.fori_loop(0, NGC, body, 0)

    for h in (0, 1):
        pltpu.make_async_copy(qrow.at[pl.ds(h * GB, GB)],
                              qd_hbm.at[pl.ds(0, GB)], sq.at[h]).wait()
        pltpu.make_async_copy(kvrow.at[pl.ds(h * GB, GB)],
                              kvj_hbm.at[pl.ds(0, GB)], skv.at[h]).wait()


def _alpha_body(qd_ref, ks_ref, ee_ref, a_ref, m_ref):
    a = jnp.sum(qd_ref[...] * (ks_ref[...] + ee_ref[...]), axis=1, keepdims=True) / 16.0
    a_ref[...] = a
    m_ref[...] = jnp.full((1, 1, 128), jnp.max(a), _f32)


def _alpha(qd, kvj, ee):
    return pl.pallas_call(
        _alpha_body,
        grid=(NEB,),
        in_specs=[
            pl.BlockSpec((BE, 256), lambda i: (i, 0)),
            pl.BlockSpec((BE, 256), lambda i: (i, 0)),
            pl.BlockSpec((BE, 256), lambda i: (i, 0)),
        ],
        out_specs=[
            pl.BlockSpec((BE, 1), lambda i: (i, 0)),
            pl.BlockSpec((1, 1, 128), lambda i: (i, 0, 0)),
        ],
        out_shape=[
            jax.ShapeDtypeStruct((E, 1), _f32),
            jax.ShapeDtypeStruct((NEB, 1, 128), _f32),
        ],
        compiler_params=pltpu.CompilerParams(dimension_semantics=("parallel",)),
    )(qd, kvj, ee)


def _y_body(c_ref, a_ref, vs_ref, ee_ref, yn_ref, ye_ref):
    ex = jnp.exp(a_ref[...] - c_ref[0, 0])
    ye_ref[...] = ex
    yn_ref[...] = (vs_ref[...] + ee_ref[...]) * ex


def _y(cmax, alpha, kvj, ee):
    return pl.pallas_call(
        _y_body,
        grid=(NEB,),
        in_specs=[
            pl.BlockSpec(memory_space=pltpu.SMEM),
            pl.BlockSpec((BE, 1), lambda i: (i, 0)),
            pl.BlockSpec((BE, 256), lambda i: (i, 1)),
            pl.BlockSpec((BE, 256), lambda i: (i, 0)),
        ],
        out_specs=[
            pl.BlockSpec((BE, 256), lambda i: (i, 0)),
            pl.BlockSpec((BE, 1), lambda i: (i, 0)),
        ],
        out_shape=[
            jax.ShapeDtypeStruct((E, 256), _f32),
            jax.ShapeDtypeStruct((E, 1), _f32),
        ],
        compiler_params=pltpu.CompilerParams(dimension_semantics=("parallel",)),
    )(cmax, alpha, kvj, ee)


@functools.cache
def _sc_scatter_fn():
    mesh = plsc.VectorSubcoreMesh(core_axis_name="c", subcore_axis_name="s")
    return functools.partial(
        pl.kernel,
        mesh=mesh,
        compiler_params=pltpu.CompilerParams(needs_layout_passes=False),
        out_type=[
            jax.ShapeDtypeStruct((NW, NR * 256), _f32),
            jax.ShapeDtypeStruct((NW, NR * 16), _f32),
        ],
        scratch_types=[
            pltpu.VMEM((SC_C,), jnp.int32),
            pltpu.VMEM((SC_C,), _f32),
            pltpu.VMEM((RCAP,), jnp.int32),
            pltpu.VMEM((RCAP,), jnp.int32),
            pltpu.VMEM((RCAP,), _f32),
            pltpu.VMEM((2 * UROW, 256), _f32),
            pltpu.VMEM((ACCR * 256,), _f32),
            pltpu.VMEM((ACCR * 16,), _f32),
            pltpu.SemaphoreType.DMA((2,)),
            pltpu.SemaphoreType.DMA,
            pltpu.SemaphoreType.DMA,
        ],
    )(_sc_scatter_body)


def _sc_scatter_body(yn_hbm, ye_hbm, dst_hbm, zn_hbm, ze_hbm, on_hbm, od_hbm,
                     dbuf, xbuf, ebuf, lbuf, cxbuf, yrow, accn, acce, sem,
                     dsem, xsem):
    c = lax.axis_index("c")
    s = lax.axis_index("s")
    w = s * NC + c
    lo = w * NR
    lane = lax.iota(jnp.int32, 16)

    pltpu.sync_copy(zn_hbm, accn)
    pltpu.sync_copy(ze_hbm, acce)

    def issue_unit(u, h):
        pltpu.async_copy(yn_hbm.at[ebuf.at[pl.ds(u * UROW, UROW)]],
                         yrow.at[pl.ds(h * UROW, UROW)], sem.at[h])

    def process_units(n_units):
        @pl.when(n_units > 0)
        def _():
            issue_unit(0, 0)

        def unit_body(u, carry2):
            par = u % 2

            @pl.when(u + 1 < n_units)
            def _():
                issue_unit(u + 1, (u + 1) % 2)

            pltpu.make_async_copy(
                yn_hbm.at[ebuf.at[pl.ds(u * UROW, UROW)]],
                yrow.at[pl.ds(par * UROW, UROW)], sem.at[par]).wait()

            def row_body(r, carry3):
                pos = jnp.full((16,), u * UROW + r, jnp.int32)
                ii = plsc.load_gather(lbuf, [pos])
                exv = plsc.load_gather(cxbuf, [pos])
                basea = ii * 256
                for l in range(16):
                    vals = yrow[par * UROW + r, pl.ds(l * 16, 16)]
                    plsc.addupdate_scatter(accn, [basea + (l * 16) + lane], vals)
                plsc.addupdate_scatter(acce, [ii * 16 + lane], exv)
                return carry3

            lax.fori_loop(0, UROW, row_body, 0)
            return carry2

        lax.fori_loop(0, n_units, unit_body, 0)

    def chunk_body(j, f):
        base = j * SC_C
        pltpu.async_copy(dst_hbm.at[pl.ds(base, SC_C)], dbuf, dsem)
        pltpu.async_copy(ye_hbm.at[pl.ds(base, SC_C)], xbuf, xsem)
        pltpu.make_async_copy(dst_hbm.at[pl.ds(base, SC_C)], dbuf, dsem).wait()
        pltpu.make_async_copy(ye_hbm.at[pl.ds(base, SC_C)], xbuf, xsem).wait()

        def scan_body(g, f2):
            d16 = dbuf[pl.ds(g * 16, 16)]
            loc = d16 - lo
            ok = (loc >= 0) & (loc < NR)
            eid = base + g * 16 + lane
            plsc.store_compressed(ebuf.at[pl.ds(f2, 16)], eid, mask=ok)
            plsc.store_compressed(lbuf.at[pl.ds(f2, 16)], loc, mask=ok)
            plsc.store_compressed(cxbuf.at[pl.ds(f2, 16)], xbuf[pl.ds(g * 16, 16)], mask=ok)
            cnt = plsc.all_reduce_population_count(ok)
            return f2 + cnt[0]

        f = lax.fori_loop(0, SC_C // 16, scan_body, f)
        n_units = f // UROW
        process_units(n_units)
        rem = f - n_units * UROW
        for t in range(UROW // 16):
            ve = ebuf[pl.ds(n_units * UROW + t * 16, 16)]
            vl = lbuf[pl.ds(n_units * UROW + t * 16, 16)]
            vx = cxbuf[pl.ds(n_units * UROW + t * 16, 16)]
            ebuf[pl.ds(t * 16, 16)] = ve
            lbuf[pl.ds(t * 16, 16)] = vl
            cxbuf[pl.ds(t * 16, 16)] = vx
        return rem

    f = lax.fori_loop(0, SC_NCH, chunk_body, 0)

    def pad_body(t, carry):
        ebuf[pl.ds(f + t * 16, 16)] = jnp.zeros((16,), jnp.int32)
        lbuf[pl.ds(f + t * 16, 16)] = jnp.full((16,), DUMP, jnp.int32)
        cxbuf[pl.ds(f + t * 16, 16)] = jnp.zeros((16,), _f32)
        return carry

    lax.fori_loop(0, UROW // 16, pad_body, 0)
    process_units((f + UROW - 1) // UROW)

    pltpu.sync_copy(accn.at[pl.ds(0, NR * 256)], on_hbm.at[w])
    pltpu.sync_copy(acce.at[pl.ds(0, NR * 16)], od_hbm.at[w])


def _pdd1_body(num_ref, den_ref, xs_ref, p_ref, x_ref, adj_ref, s1_ref, s2_ref):
    den = jnp.max(den_ref[...], axis=1, keepdims=True)
    x = num_ref[...] * (1.0 / (den + 1e-16)) + xs_ref[...]
    adj = p_ref[...] + x
    x_ref[...] = x
    adj_ref[...] = adj
    s1_ref[...] = jnp.sum(adj, axis=0, keepdims=True).reshape(1, 1, 256)
    s2_ref[...] = jnp.sum(adj * adj, axis=0, keepdims=True).reshape(1, 1, 256)


def _pdd1(num, den, xs, p):
    return pl.pallas_call(
        _pdd1_body,
        grid=(NNB,),
        in_specs=[
            pl.BlockSpec((BN, 256), lambda i: (i, 0)),
            pl.BlockSpec((BN, 16), lambda i: (i, 0)),
            pl.BlockSpec((BN, 256), lambda i: (i, 0)),
            pl.BlockSpec((BN, 256), lambda i: (i, 0)),
        ],
        out_specs=[
            pl.BlockSpec((BN, 256), lambda i: (i, 0)),
            pl.BlockSpec((BN, 256), lambda i: (i, 0)),
            pl.BlockSpec((1, 1, 256), lambda i: (i, 0, 0)),
            pl.BlockSpec((1, 1, 256), lambda i: (i, 0, 0)),
        ],
        out_shape=[
            jax.ShapeDtypeStruct((N, 256), _f32),
            jax.ShapeDtypeStruct((N, 256), _f32),
            jax.ShapeDtypeStruct((NNB, 1, 256), _f32),
            jax.ShapeDtypeStruct((NNB, 1, 256), _f32),
        ],
        compiler_params=pltpu.CompilerParams(dimension_semantics=("parallel",)),
    )(num, den, xs, p)


def _pdd2_body(adj_ref, x_ref, sc_ref, sh_ref, w1_ref, b1_ref, w2_ref, b2_ref,
               w3_ref, b3_ref, o_ref):
    h = adj_ref[...] * sc_ref[...] + sh_ref[...]
    h2 = jnp.dot(h, w1_ref[...], preferred_element_type=_f32) + b1_ref[...]
    x1 = h2[:, :256]
    x2 = h2[:, 256:]
    x1 = jnp.dot(x1, w2_ref[...], preferred_element_type=_f32) + b2_ref[...]
    x2 = 0.5 * x2 * (1.0 + lax.erf(x2 * 0.7071067811865476))
    o_ref[...] = (jnp.dot(x1 * x2, w3_ref[...], preferred_element_type=_f32)
                  + b3_ref[...] + x_ref[...])


def _pdd2(adj, x, scale, shift, w1, b1, w2, b2, w3, b3):
    return pl.pallas_call(
        _pdd2_body,
        grid=(NNB,),
        in_specs=[
            pl.BlockSpec((BN, 256), lambda i: (i, 0)),
            pl.BlockSpec((BN, 256), lambda i: (i, 0)),
            pl.BlockSpec((1, 256), lambda i: (0, 0)),
            pl.BlockSpec((1, 256), lambda i: (0, 0)),
            pl.BlockSpec((256, 512), lambda i: (0, 0)),
            pl.BlockSpec((1, 512), lambda i: (0, 0)),
            pl.BlockSpec((256, 256), lambda i: (0, 0)),
            pl.BlockSpec((1, 256), lambda i: (0, 0)),
            pl.BlockSpec((256, 256), lambda i: (0, 0)),
            pl.BlockSpec((1, 256), lambda i: (0, 0)),
        ],
        out_specs=pl.BlockSpec((BN, 256), lambda i: (i, 0)),
        out_shape=jax.ShapeDtypeStruct((N, 256), _f32),
        compiler_params=pltpu.CompilerParams(dimension_semantics=("parallel",)),
    )(adj, x, scale, shift, w1, b1, w2, b2, w3, b3)


def _pool_body(num_ref, den_ref, xs_ref, b_ref, sums_ref, cnt_ref):
    i = pl.program_id(0)
    den = jnp.max(den_ref[...], axis=1, keepdims=True)
    x3 = num_ref[...] * (1.0 / (den + 1e-16)) + xs_ref[...]
    b = jnp.max(b_ref[...], axis=1, keepdims=True)
    ids = lax.broadcasted_iota(jnp.int32, (1, 128), 1).astype(_f32)
    onehot = (b == ids).astype(_f32)
    part = lax.dot_general(onehot, x3, (((0,), (0,)), ((), ())),
                           preferred_element_type=_f32)
    cpart = jnp.sum(onehot, axis=0, keepdims=True)

    @pl.when(i == 0)
    def _():
        sums_ref[...] = jnp.zeros_like(sums_ref)
        cnt_ref[...] = jnp.zeros_like(cnt_ref)

    sums_ref[...] += part
    cnt_ref[...] += jnp.broadcast_to(cpart, (8, 128))


def _pool(num, den, xs, batchf):
    return pl.pallas_call(
        _pool_body,
        grid=(NNB,),
        in_specs=[
            pl.BlockSpec((BN, 256), lambda i: (i, 0)),
            pl.BlockSpec((BN, 16), lambda i: (i, 0)),
            pl.BlockSpec((BN, 256), lambda i: (i, 0)),
            pl.BlockSpec((BN, 16), lambda i: (i, 0)),
        ],
        out_specs=[
            pl.BlockSpec((128, 256), lambda i: (0, 0)),
            pl.BlockSpec((8, 128), lambda i: (0, 0)),
        ],
        out_shape=[
            jax.ShapeDtypeStruct((128, 256), _f32),
            jax.ShapeDtypeStruct((8, 128), _f32),
        ],
        compiler_params=pltpu.CompilerParams(dimension_semantics=("arbitrary",)),
    )(num, den, xs, batchf)


def _head_body(p_ref, wf_ref, bf_ref, wo_ref, bo_ref, o_ref):
    pooled = p_ref[...]
    f = pooled + _silu(jnp.dot(pooled, wf_ref[...], preferred_element_type=_f32)
                       + bf_ref[...])
    o_ref[...] = jnp.dot(f, wo_ref[...], preferred_element_type=_f32) + bo_ref[...]


def _head(pooled, wf, bf, wop, bop):
    return pl.pallas_call(
        _head_body,
        in_specs=[
            pl.BlockSpec((64, 256), lambda: (0, 0)),
            pl.BlockSpec((256, 256), lambda: (0, 0)),
            pl.BlockSpec((1, 256), lambda: (0, 0)),
            pl.BlockSpec((256, 128), lambda: (0, 0)),
            pl.BlockSpec((1, 128), lambda: (0, 0)),
        ],
        out_specs=pl.BlockSpec((64, 128), lambda: (0, 0)),
        out_shape=jax.ShapeDtypeStruct((64, 128), _f32),
    )(pooled, wf, bf, wop, bop)


def _conv_sparse(q, kv, ee, src, dst):
    qd, kvj = _sc_gather_fn()(src, dst, q, kv)
    alpha, bmax = _alpha(qd, kvj, ee)
    cmax = jnp.max(bmax).reshape(1, 1)
    ynum, yex = _y(cmax, alpha, kvj, ee)
    yex = yex.reshape(E)
    zn = jnp.zeros((ACCR * 256,), _f32)
    ze = jnp.zeros((ACCR * 16,), _f32)
    onum, oden = _sc_scatter_fn()(ynum, yex, dst, zn, ze)
    num = onum.reshape(NW * NR, 256)[:N]
    den = oden.reshape(NW * NR, 16)[:N]
    return num, den


def kernel(node, edge_attr, pdd, edge_index, batch, Wa1, ba1, Wa2, ba2,
           Wemb, bemb, Wp, bp, Wq, bq, Wk, bk, Wv, bv, Wed, bed, Ws, bs,
           W1p, b1p, W2p, b2p, W3p, b3p, gbn, bbn, Wf, bf, Wo, bo):
    ef = -1.0 / jnp.linalg.norm(edge_attr, axis=1)
    ef3d = ef.reshape(E, 1)
    wedcat = jnp.concatenate([Wed[0], Wed[1], Wed[2]], axis=1)
    bedcat = jnp.concatenate([bed[0], bed[1], bed[2]], axis=0).reshape(1, 768)
    ee0, ee1, ee2 = _embed(ef3d, Wemb, bemb.reshape(1, 256), wedcat, bedcat)
    ees = (ee0, ee1, ee2)

    nodep = jnp.pad(node, ((0, 0), (0, 128 - node.shape[1])))
    pddp = jnp.pad(pdd, ((0, 0), (0, 128 - pdd.shape[1])))
    wa1p = jnp.pad(Wa1, ((0, 128 - Wa1.shape[0]), (0, 0)))
    wpp = jnp.pad(Wp, ((0, 128 - Wp.shape[0]), (0, 0)))
    x, p = _x0p0(nodep, pddp, wa1p, ba1.reshape(1, 256), Wa2,
                 ba2.reshape(1, 256), wpp, bp.reshape(1, 256))

    src = edge_index[0]
    dst = edge_index[1]

    for c in range(3):
        wcat = jnp.concatenate([Wq[c], Wk[c], Wv[c], Ws[c]], axis=1)
        bcat = jnp.concatenate([bq[c], bk[c], bv[c], bs[c]], axis=0).reshape(1, 1024)
        q, kv, xs = _qkvs(x, wcat, bcat)
        num, den = _conv_sparse(q, kv, ees[c], src, dst)
        if c < 2:
            xc, adj, s1, s2 = _pdd1(num, den, xs, p)
            mu = jnp.sum(s1, axis=(0, 1)) / N
            var = jnp.sum(s2, axis=(0, 1)) / N - mu * mu
            scale = gbn[c] / jnp.sqrt(var + 1e-5)
            shift = bbn[c] - mu * scale
            x = _pdd2(adj, xc, scale.reshape(1, 256), shift.reshape(1, 256),
                      W1p[c], b1p[c].reshape(1, 512), W2p[c], b2p[c].reshape(1, 256),
                      W3p[c], b3p[c].reshape(1, 256))
            p = adj
        else:
            batchf = jnp.broadcast_to(batch.astype(_f32)[:, None], (N, 16))
            sums, cnt = _pool(num, den, xs, batchf)
            pooled = sums[:G] / jnp.maximum(cnt[0, :G], 1.0)[:, None]
            wop = jnp.pad(Wo, ((0, 0), (0, 127)))
            bop = jnp.pad(bo, ((0, 127))).reshape(1, 128)
            res = _head(pooled, Wf, bf.reshape(1, 256), wop, bop)
            return res[:, 0]


# speedup vs baseline: 2.4008x; 1.0290x over previous
import functools

import jax
import jax.numpy as jnp
from jax import lax
from jax.experimental import pallas as pl
from jax.experimental.pallas import tpu as pltpu
from jax.experimental.pallas import tpu_sc as plsc

N = 10000
E = 160000
G = 64
D = 256

BE = 640
NEB = E // BE
BN = 400
NNB = N // BN

NC = 2
NS = 16
NW = NC * NS
CE = E // NW
GB = 40
NGC = CE // GB
SC_C = 1600
SC_NCH = E // SC_C
NR = 320
DUMP = NR
ACCR = NR + 1
UROW = 48
RCAP = SC_C + 2 * UROW
_f32 = jnp.float32


def _silu(x):
    return x * jax.nn.sigmoid(x)


def _embed_body(ef_ref, wemb_ref, bemb_ref, wed_ref, bed_ref, e0_ref, e1_ref, e2_ref):
    ef = ef_ref[...]
    centers = -6.0 + (6.0 / 255.0) * lax.broadcasted_iota(jnp.int32, (1, 256), 1).astype(_f32)
    gamma = 1.0 / (6.0 / 255.0)
    diff = ef - centers
    rbf = jnp.exp(-gamma * diff * diff)
    z = jnp.dot(rbf, wemb_ref[...], preferred_element_type=_f32) + bemb_ref[...]
    e = _silu(z)
    ee = jnp.dot(e, wed_ref[...], preferred_element_type=_f32) + bed_ref[...]
    e0_ref[...] = ee[:, :256]
    e1_ref[...] = ee[:, 256:512]
    e2_ref[...] = ee[:, 512:]


def _embed(ef3d, wemb, bemb, wedcat, bedcat):
    return pl.pallas_call(
        _embed_body,
        grid=(NEB,),
        in_specs=[
            pl.BlockSpec((BE, 1), lambda i: (i, 0)),
            pl.BlockSpec((256, 256), lambda i: (0, 0)),
            pl.BlockSpec((1, 256), lambda i: (0, 0)),
            pl.BlockSpec((256, 768), lambda i: (0, 0)),
            pl.BlockSpec((1, 768), lambda i: (0, 0)),
        ],
        out_specs=[pl.BlockSpec((BE, 256), lambda i: (i, 0))] * 3,
        out_shape=[jax.ShapeDtypeStruct((E, 256), _f32)] * 3,
        compiler_params=pltpu.CompilerParams(dimension_semantics=("parallel",)),
    )(ef3d, wemb, bemb, wedcat, bedcat)


def _x0p0_body(node_ref, pdd_ref, wa1_ref, ba1_ref, wa2_ref, ba2_ref,
               wp_ref, bp_ref, x0_ref, p0_ref):
    h = jnp.dot(node_ref[...], wa1_ref[...], preferred_element_type=_f32) + ba1_ref[...]
    h = _silu(h)
    x0_ref[...] = jnp.dot(h, wa2_ref[...], preferred_element_type=_f32) + ba2_ref[...]
    p0_ref[...] = jnp.dot(pdd_ref[...], wp_ref[...], preferred_element_type=_f32) + bp_ref[...]


def _x0p0(nodep, pddp, wa1p, ba1, wa2, ba2, wpp, bp):
    return pl.pallas_call(
        _x0p0_body,
        grid=(NNB,),
        in_specs=[
            pl.BlockSpec((BN, 128), lambda i: (i, 0)),
            pl.BlockSpec((BN, 128), lambda i: (i, 0)),
            pl.BlockSpec((128, 256), lambda i: (0, 0)),
            pl.BlockSpec((1, 256), lambda i: (0, 0)),
            pl.BlockSpec((256, 256), lambda i: (0, 0)),
            pl.BlockSpec((1, 256), lambda i: (0, 0)),
            pl.BlockSpec((128, 256), lambda i: (0, 0)),
            pl.BlockSpec((1, 256), lambda i: (0, 0)),
        ],
        out_specs=[pl.BlockSpec((BN, 256), lambda i: (i, 0))] * 2,
        out_shape=[jax.ShapeDtypeStruct((N, 256), _f32)] * 2,
        compiler_params=pltpu.CompilerParams(dimension_semantics=("parallel",)),
    )(nodep, pddp, wa1p, ba1, wa2, ba2, wpp, bp)


def _qkvs_body(x_ref, w_ref, b_ref, q_ref, kv_ref, s_ref):
    o = jnp.dot(x_ref[...], w_ref[...], preferred_element_type=_f32) + b_ref[...]
    q_ref[...] = o[:, :256]
    kv_ref[...] = o[:, 256:768]
    s_ref[...] = o[:, 768:]


def _qkvs(x, wcat, bcat):
    return pl.pallas_call(
        _qkvs_body,
        grid=(NNB,),
        in_specs=[
            pl.BlockSpec((BN, 256), lambda i: (i, 0)),
            pl.BlockSpec((256, 1024), lambda i: (0, 0)),
            pl.BlockSpec((1, 1024), lambda i: (0, 0)),
        ],
        out_specs=[
            pl.BlockSpec((BN, 256), lambda i: (i, 0)),
            pl.BlockSpec((BN, 512), lambda i: (i, 0)),
            pl.BlockSpec((BN, 256), lambda i: (i, 0)),
        ],
        out_shape=[
            jax.ShapeDtypeStruct((N, 256), _f32),
            jax.ShapeDtypeStruct((N, 512), _f32),
            jax.ShapeDtypeStruct((N, 256), _f32),
        ],
        compiler_params=pltpu.CompilerParams(dimension_semantics=("parallel",)),
    )(x, wcat, bcat)


@functools.cache
def _sc_gather_fn():
    mesh = plsc.VectorSubcoreMesh(core_axis_name="c", subcore_axis_name="s")
    return functools.partial(
        pl.kernel,
        mesh=mesh,
        out_type=[
            jax.ShapeDtypeStruct((E, 256), _f32),
            jax.ShapeDtypeStruct((E, 512), _f32),
        ],
        scratch_types=[
            pltpu.VMEM((2 * GB,), jnp.int32),
            pltpu.VMEM((2 * GB,), jnp.int32),
            pltpu.VMEM((2 * GB, 256), _f32),
            pltpu.VMEM((2 * GB, 512), _f32),
            pltpu.SemaphoreType.DMA((2,)),
            pltpu.SemaphoreType.DMA((2,)),
            pltpu.SemaphoreType.DMA((2,)),
            pltpu.SemaphoreType.DMA((2,)),
        ],
    )(_sc_gather_body)


def _sc_gather_body(src_hbm, dst_hbm, q_hbm, kv_hbm,
                    qd_hbm, kvj_hbm, srci, dsti, qrow, kvrow,
                    gq, gkv, sq, skv):
    c = lax.axis_index("c")
    s = lax.axis_index("s")
    wid = s * NC + c
    base0 = wid * CE

    def load_and_gather(j, h):
        base = base0 + j * GB
        pltpu.sync_copy(src_hbm.at[pl.ds(base, GB)], srci.at[pl.ds(h * GB, GB)])
        pltpu.sync_copy(dst_hbm.at[pl.ds(base, GB)], dsti.at[pl.ds(h * GB, GB)])
        pltpu.async_copy(q_hbm.at[dsti.at[pl.ds(h * GB, GB)]],
                         qrow.at[pl.ds(h * GB, GB)], gq.at[h])
        pltpu.async_copy(kv_hbm.at[srci.at[pl.ds(h * GB, GB)]],
                         kvrow.at[pl.ds(h * GB, GB)], gkv.at[h])

    load_and_gather(0, 0)

    def body(j, carry):
        par = j % 2
        nxt = (j + 1) % 2

        @pl.when(j + 1 < NGC)
        def _():
            @pl.when(j >= 1)
            def _():
                pltpu.make_async_copy(qrow.at[pl.ds(nxt * GB, GB)],
                                      qd_hbm.at[pl.ds(0, GB)], sq.at[nxt]).wait()
                pltpu.make_async_copy(kvrow.at[pl.ds(nxt * GB, GB)],
                                      kvj_hbm.at[pl.ds(0, GB)], skv.at[nxt]).wait()

            load_and_gather(j + 1, nxt)

        base = base0 + j * GB
        pltpu.make_async_copy(q_hbm.at[dsti.at[pl.ds(par * GB, GB)]],
                              qrow.at[pl.ds(par * GB, GB)], gq.at[par]).wait()
        pltpu.make_async_copy(kv_hbm.at[srci.at[pl.ds(par * GB, GB)]],
                              kvrow.at[pl.ds(par * GB, GB)], gkv.at[par]).wait()
        pltpu.async_copy(qrow.at[pl.ds(par * GB, GB)],
                         qd_hbm.at[pl.ds(base, GB)], sq.at[par])
        pltpu.async_copy(kvrow.at[pl.ds(par * GB, GB)],
                         kvj_hbm.at[pl.ds(base, GB)], skv.at[par])
        return carry

    lax.fori_loop(0, NGC, body, 0)

    for h in (0, 1):
        pltpu.make_async_copy(qrow.at[pl.ds(h * GB, GB)],
                              qd_hbm.at[pl.ds(0, GB)], sq.at[h]).wait()
        pltpu.make_async_copy(kvrow.at[pl.ds(h * GB, GB)],
                              kvj_hbm.at[pl.ds(0, GB)], skv.at[h]).wait()


def _alpha_body(qd_ref, ks_ref, ee_ref, a_ref, m_ref):
    a = jnp.sum(qd_ref[...] * (ks_ref[...] + ee_ref[...]), axis=1, keepdims=True) / 16.0
    a_ref[...] = a
    m_ref[...] = jnp.full((1, 1, 128), jnp.max(a), _f32)


def _alpha(qd, kvj, ee):
    return pl.pallas_call(
        _alpha_body,
        grid=(NEB,),
        in_specs=[
            pl.BlockSpec((BE, 256), lambda i: (i, 0)),
            pl.BlockSpec((BE, 256), lambda i: (i, 0)),
            pl.BlockSpec((BE, 256), lambda i: (i, 0)),
        ],
        out_specs=[
            pl.BlockSpec((BE, 1), lambda i: (i, 0)),
            pl.BlockSpec((1, 1, 128), lambda i: (i, 0, 0)),
        ],
        out_shape=[
            jax.ShapeDtypeStruct((E, 1), _f32),
            jax.ShapeDtypeStruct((NEB, 1, 128), _f32),
        ],
        compiler_params=pltpu.CompilerParams(dimension_semantics=("parallel",)),
    )(qd, kvj, ee)


def _y_body(c_ref, a_ref, vs_ref, ee_ref, yn_ref, ye_ref):
    ex = jnp.exp(a_ref[...] - c_ref[0, 0])
    ye_ref[...] = ex
    yn_ref[...] = (vs_ref[...] + ee_ref[...]) * ex


def _y(cmax, alpha, kvj, ee):
    return pl.pallas_call(
        _y_body,
        grid=(NEB,),
        in_specs=[
            pl.BlockSpec(memory_space=pltpu.SMEM),
            pl.BlockSpec((BE, 1), lambda i: (i, 0)),
            pl.BlockSpec((BE, 256), lambda i: (i, 1)),
            pl.BlockSpec((BE, 256), lambda i: (i, 0)),
        ],
        out_specs=[
            pl.BlockSpec((BE, 256), lambda i: (i, 0)),
            pl.BlockSpec((BE, 1), lambda i: (i, 0)),
        ],
        out_shape=[
            jax.ShapeDtypeStruct((E, 256), _f32),
            jax.ShapeDtypeStruct((E, 1), _f32),
        ],
        compiler_params=pltpu.CompilerParams(dimension_semantics=("parallel",)),
    )(cmax, alpha, kvj, ee)


@functools.cache
def _sc_scatter_fn():
    mesh = plsc.VectorSubcoreMesh(core_axis_name="c", subcore_axis_name="s")
    return functools.partial(
        pl.kernel,
        mesh=mesh,
        compiler_params=pltpu.CompilerParams(needs_layout_passes=False),
        out_type=[
            jax.ShapeDtypeStruct((NW, NR * 256), _f32),
            jax.ShapeDtypeStruct((NW, NR * 16), _f32),
        ],
        scratch_types=[
            pltpu.VMEM((2 * SC_C,), jnp.int32),
            pltpu.VMEM((2 * SC_C,), _f32),
            pltpu.VMEM((RCAP,), jnp.int32),
            pltpu.VMEM((RCAP,), jnp.int32),
            pltpu.VMEM((RCAP,), _f32),
            pltpu.VMEM((2 * UROW, 256), _f32),
            pltpu.VMEM((ACCR * 256,), _f32),
            pltpu.VMEM((ACCR * 16,), _f32),
            pltpu.SemaphoreType.DMA((2,)),
            pltpu.SemaphoreType.DMA((2,)),
            pltpu.SemaphoreType.DMA((2,)),
        ],
    )(_sc_scatter_body)


def _sc_scatter_body(yn_hbm, ye_hbm, dst_hbm, zn_hbm, ze_hbm, on_hbm, od_hbm,
                     dbuf, xbuf, ebuf, lbuf, cxbuf, yrow, accn, acce, sem,
                     dsem, xsem):
    c = lax.axis_index("c")
    s = lax.axis_index("s")
    w = s * NC + c
    lo = w * NR
    lane = lax.iota(jnp.int32, 16)

    pltpu.sync_copy(zn_hbm, accn)
    pltpu.sync_copy(ze_hbm, acce)

    def issue_unit(u, h):
        pltpu.async_copy(yn_hbm.at[ebuf.at[pl.ds(u * UROW, UROW)]],
                         yrow.at[pl.ds(h * UROW, UROW)], sem.at[h])

    def process_units(n_units):
        @pl.when(n_units > 0)
        def _():
            issue_unit(0, 0)

        def unit_body(u, carry2):
            par = u % 2

            @pl.when(u + 1 < n_units)
            def _():
                issue_unit(u + 1, (u + 1) % 2)

            pltpu.make_async_copy(
                yn_hbm.at[ebuf.at[pl.ds(u * UROW, UROW)]],
                yrow.at[pl.ds(par * UROW, UROW)], sem.at[par]).wait()

            def row_body(r, carry3):
                pos = jnp.full((16,), u * UROW + r, jnp.int32)
                ii = plsc.load_gather(lbuf, [pos])
                exv = plsc.load_gather(cxbuf, [pos])
                basea = ii * 256
                for l in range(16):
                    vals = yrow[par * UROW + r, pl.ds(l * 16, 16)]
                    plsc.addupdate_scatter(accn, [basea + (l * 16) + lane], vals)
                plsc.addupdate_scatter(acce, [ii * 16 + lane], exv)
                return carry3

            lax.fori_loop(0, UROW, row_body, 0)
            return carry2

        lax.fori_loop(0, n_units, unit_body, 0)

    def load_chunk(j, h):
        base = j * SC_C
        pltpu.async_copy(dst_hbm.at[pl.ds(base, SC_C)], dbuf.at[pl.ds(h * SC_C, SC_C)],
                         dsem.at[h])
        pltpu.async_copy(ye_hbm.at[pl.ds(base, SC_C)], xbuf.at[pl.ds(h * SC_C, SC_C)],
                         xsem.at[h])

    load_chunk(0, 0)

    def chunk_body(j, f):
        par = j % 2
        nxt = (j + 1) % 2

        @pl.when(j + 1 < SC_NCH)
        def _():
            load_chunk(j + 1, nxt)

        base = j * SC_C
        pltpu.make_async_copy(dst_hbm.at[pl.ds(base, SC_C)],
                              dbuf.at[pl.ds(par * SC_C, SC_C)], dsem.at[par]).wait()
        pltpu.make_async_copy(ye_hbm.at[pl.ds(base, SC_C)],
                              xbuf.at[pl.ds(par * SC_C, SC_C)], xsem.at[par]).wait()

        def scan_body(g, f2):
            d16 = dbuf[pl.ds(par * SC_C + g * 16, 16)]
            loc = d16 - lo
            ok = (loc >= 0) & (loc < NR)
            eid = base + g * 16 + lane
            plsc.store_compressed(ebuf.at[pl.ds(f2, 16)], eid, mask=ok)
            plsc.store_compressed(lbuf.at[pl.ds(f2, 16)], loc, mask=ok)
            plsc.store_compressed(cxbuf.at[pl.ds(f2, 16)],
                                  xbuf[pl.ds(par * SC_C + g * 16, 16)], mask=ok)
            cnt = plsc.all_reduce_population_count(ok)
            return f2 + cnt[0]

        f = lax.fori_loop(0, SC_C // 16, scan_body, f)
        n_units = f // UROW
        process_units(n_units)
        rem = f - n_units * UROW
        for t in range(UROW // 16):
            ve = ebuf[pl.ds(n_units * UROW + t * 16, 16)]
            vl = lbuf[pl.ds(n_units * UROW + t * 16, 16)]
            vx = cxbuf[pl.ds(n_units * UROW + t * 16, 16)]
            ebuf[pl.ds(t * 16, 16)] = ve
            lbuf[pl.ds(t * 16, 16)] = vl
            cxbuf[pl.ds(t * 16, 16)] = vx
        return rem

    f = lax.fori_loop(0, SC_NCH, chunk_body, 0)

    def pad_body(t, carry):
        ebuf[pl.ds(f + t * 16, 16)] = jnp.zeros((16,), jnp.int32)
        lbuf[pl.ds(f + t * 16, 16)] = jnp.full((16,), DUMP, jnp.int32)
        cxbuf[pl.ds(f + t * 16, 16)] = jnp.zeros((16,), _f32)
        return carry

    lax.fori_loop(0, UROW // 16, pad_body, 0)
    process_units((f + UROW - 1) // UROW)

    pltpu.sync_copy(accn.at[pl.ds(0, NR * 256)], on_hbm.at[w])
    pltpu.sync_copy(acce.at[pl.ds(0, NR * 16)], od_hbm.at[w])


def _pdd1_body(num_ref, den_ref, xs_ref, p_ref, x_ref, adj_ref, s1_ref, s2_ref):
    den = jnp.max(den_ref[...], axis=1, keepdims=True)
    x = num_ref[...] * (1.0 / (den + 1e-16)) + xs_ref[...]
    adj = p_ref[...] + x
    x_ref[...] = x
    adj_ref[...] = adj
    s1_ref[...] = jnp.sum(adj, axis=0, keepdims=True).reshape(1, 1, 256)
    s2_ref[...] = jnp.sum(adj * adj, axis=0, keepdims=True).reshape(1, 1, 256)


def _pdd1(num, den, xs, p):
    return pl.pallas_call(
        _pdd1_body,
        grid=(NNB,),
        in_specs=[
            pl.BlockSpec((BN, 256), lambda i: (i, 0)),
            pl.BlockSpec((BN, 16), lambda i: (i, 0)),
            pl.BlockSpec((BN, 256), lambda i: (i, 0)),
            pl.BlockSpec((BN, 256), lambda i: (i, 0)),
        ],
        out_specs=[
            pl.BlockSpec((BN, 256), lambda i: (i, 0)),
            pl.BlockSpec((BN, 256), lambda i: (i, 0)),
            pl.BlockSpec((1, 1, 256), lambda i: (i, 0, 0)),
            pl.BlockSpec((1, 1, 256), lambda i: (i, 0, 0)),
        ],
        out_shape=[
            jax.ShapeDtypeStruct((N, 256), _f32),
            jax.ShapeDtypeStruct((N, 256), _f32),
            jax.ShapeDtypeStruct((NNB, 1, 256), _f32),
            jax.ShapeDtypeStruct((NNB, 1, 256), _f32),
        ],
        compiler_params=pltpu.CompilerParams(dimension_semantics=("parallel",)),
    )(num, den, xs, p)


def _pdd2_body(adj_ref, x_ref, sc_ref, sh_ref, w1_ref, b1_ref, w2_ref, b2_ref,
               w3_ref, b3_ref, o_ref):
    h = adj_ref[...] * sc_ref[...] + sh_ref[...]
    h2 = jnp.dot(h, w1_ref[...], preferred_element_type=_f32) + b1_ref[...]
    x1 = h2[:, :256]
    x2 = h2[:, 256:]
    x1 = jnp.dot(x1, w2_ref[...], preferred_element_type=_f32) + b2_ref[...]
    x2 = 0.5 * x2 * (1.0 + lax.erf(x2 * 0.7071067811865476))
    o_ref[...] = (jnp.dot(x1 * x2, w3_ref[...], preferred_element_type=_f32)
                  + b3_ref[...] + x_ref[...])


def _pdd2(adj, x, scale, shift, w1, b1, w2, b2, w3, b3):
    return pl.pallas_call(
        _pdd2_body,
        grid=(NNB,),
        in_specs=[
            pl.BlockSpec((BN, 256), lambda i: (i, 0)),
            pl.BlockSpec((BN, 256), lambda i: (i, 0)),
            pl.BlockSpec((1, 256), lambda i: (0, 0)),
            pl.BlockSpec((1, 256), lambda i: (0, 0)),
            pl.BlockSpec((256, 512), lambda i: (0, 0)),
            pl.BlockSpec((1, 512), lambda i: (0, 0)),
            pl.BlockSpec((256, 256), lambda i: (0, 0)),
            pl.BlockSpec((1, 256), lambda i: (0, 0)),
            pl.BlockSpec((256, 256), lambda i: (0, 0)),
            pl.BlockSpec((1, 256), lambda i: (0, 0)),
        ],
        out_specs=pl.BlockSpec((BN, 256), lambda i: (i, 0)),
        out_shape=jax.ShapeDtypeStruct((N, 256), _f32),
        compiler_params=pltpu.CompilerParams(dimension_semantics=("parallel",)),
    )(adj, x, scale, shift, w1, b1, w2, b2, w3, b3)


def _pool_body(num_ref, den_ref, xs_ref, b_ref, sums_ref, cnt_ref):
    i = pl.program_id(0)
    den = jnp.max(den_ref[...], axis=1, keepdims=True)
    x3 = num_ref[...] * (1.0 / (den + 1e-16)) + xs_ref[...]
    b = jnp.max(b_ref[...], axis=1, keepdims=True)
    ids = lax.broadcasted_iota(jnp.int32, (1, 128), 1).astype(_f32)
    onehot = (b == ids).astype(_f32)
    part = lax.dot_general(onehot, x3, (((0,), (0,)), ((), ())),
                           preferred_element_type=_f32)
    cpart = jnp.sum(onehot, axis=0, keepdims=True)

    @pl.when(i == 0)
    def _():
        sums_ref[...] = jnp.zeros_like(sums_ref)
        cnt_ref[...] = jnp.zeros_like(cnt_ref)

    sums_ref[...] += part
    cnt_ref[...] += jnp.broadcast_to(cpart, (8, 128))


def _pool(num, den, xs, batchf):
    return pl.pallas_call(
        _pool_body,
        grid=(NNB,),
        in_specs=[
            pl.BlockSpec((BN, 256), lambda i: (i, 0)),
            pl.BlockSpec((BN, 16), lambda i: (i, 0)),
            pl.BlockSpec((BN, 256), lambda i: (i, 0)),
            pl.BlockSpec((BN, 16), lambda i: (i, 0)),
        ],
        out_specs=[
            pl.BlockSpec((128, 256), lambda i: (0, 0)),
            pl.BlockSpec((8, 128), lambda i: (0, 0)),
        ],
        out_shape=[
            jax.ShapeDtypeStruct((128, 256), _f32),
            jax.ShapeDtypeStruct((8, 128), _f32),
        ],
        compiler_params=pltpu.CompilerParams(dimension_semantics=("arbitrary",)),
    )(num, den, xs, batchf)


def _head_body(p_ref, wf_ref, bf_ref, wo_ref, bo_ref, o_ref):
    pooled = p_ref[...]
    f = pooled + _silu(jnp.dot(pooled, wf_ref[...], preferred_element_type=_f32)
                       + bf_ref[...])
    o_ref[...] = jnp.dot(f, wo_ref[...], preferred_element_type=_f32) + bo_ref[...]


def _head(pooled, wf, bf, wop, bop):
    return pl.pallas_call(
        _head_body,
        in_specs=[
            pl.BlockSpec((64, 256), lambda: (0, 0)),
            pl.BlockSpec((256, 256), lambda: (0, 0)),
            pl.BlockSpec((1, 256), lambda: (0, 0)),
            pl.BlockSpec((256, 128), lambda: (0, 0)),
            pl.BlockSpec((1, 128), lambda: (0, 0)),
        ],
        out_specs=pl.BlockSpec((64, 128), lambda: (0, 0)),
        out_shape=jax.ShapeDtypeStruct((64, 128), _f32),
    )(pooled, wf, bf, wop, bop)


def _conv_sparse(q, kv, ee, src, dst):
    qd, kvj = _sc_gather_fn()(src, dst, q, kv)
    alpha, bmax = _alpha(qd, kvj, ee)
    cmax = jnp.max(bmax).reshape(1, 1)
    ynum, yex = _y(cmax, alpha, kvj, ee)
    yex = yex.reshape(E)
    zn = jnp.zeros((ACCR * 256,), _f32)
    ze = jnp.zeros((ACCR * 16,), _f32)
    onum, oden = _sc_scatter_fn()(ynum, yex, dst, zn, ze)
    num = onum.reshape(NW * NR, 256)[:N]
    den = oden.reshape(NW * NR, 16)[:N]
    return num, den


def kernel(node, edge_attr, pdd, edge_index, batch, Wa1, ba1, Wa2, ba2,
           Wemb, bemb, Wp, bp, Wq, bq, Wk, bk, Wv, bv, Wed, bed, Ws, bs,
           W1p, b1p, W2p, b2p, W3p, b3p, gbn, bbn, Wf, bf, Wo, bo):
    ef = -1.0 / jnp.linalg.norm(edge_attr, axis=1)
    ef3d = ef.reshape(E, 1)
    wedcat = jnp.concatenate([Wed[0], Wed[1], Wed[2]], axis=1)
    bedcat = jnp.concatenate([bed[0], bed[1], bed[2]], axis=0).reshape(1, 768)
    ee0, ee1, ee2 = _embed(ef3d, Wemb, bemb.reshape(1, 256), wedcat, bedcat)
    ees = (ee0, ee1, ee2)

    nodep = jnp.pad(node, ((0, 0), (0, 128 - node.shape[1])))
    pddp = jnp.pad(pdd, ((0, 0), (0, 128 - pdd.shape[1])))
    wa1p = jnp.pad(Wa1, ((0, 128 - Wa1.shape[0]), (0, 0)))
    wpp = jnp.pad(Wp, ((0, 128 - Wp.shape[0]), (0, 0)))
    x, p = _x0p0(nodep, pddp, wa1p, ba1.reshape(1, 256), Wa2,
                 ba2.reshape(1, 256), wpp, bp.reshape(1, 256))

    src = edge_index[0]
    dst = edge_index[1]

    for c in range(3):
        wcat = jnp.concatenate([Wq[c], Wk[c], Wv[c], Ws[c]], axis=1)
        bcat = jnp.concatenate([bq[c], bk[c], bv[c], bs[c]], axis=0).reshape(1, 1024)
        q, kv, xs = _qkvs(x, wcat, bcat)
        num, den = _conv_sparse(q, kv, ees[c], src, dst)
        if c < 2:
            xc, adj, s1, s2 = _pdd1(num, den, xs, p)
            mu = jnp.sum(s1, axis=(0, 1)) / N
            var = jnp.sum(s2, axis=(0, 1)) / N - mu * mu
            scale = gbn[c] / jnp.sqrt(var + 1e-5)
            shift = bbn[c] - mu * scale
            x = _pdd2(adj, xc, scale.reshape(1, 256), shift.reshape(1, 256),
                      W1p[c], b1p[c].reshape(1, 512), W2p[c], b2p[c].reshape(1, 256),
                      W3p[c], b3p[c].reshape(1, 256))
            p = adj
        else:
            batchf = jnp.broadcast_to(batch.astype(_f32)[:, None], (N, 16))
            sums, cnt = _pool(num, den, xs, batchf)
            pooled = sums[:G] / jnp.maximum(cnt[0, :G], 1.0)[:, None]
            wop = jnp.pad(Wo, ((0, 0), (0, 127)))
            bop = jnp.pad(bo, ((0, 127))).reshape(1, 128)
            res = _head(pooled, Wf, bf.reshape(1, 256), wop, bop)
            return res[:, 0]
